# Initial kernel scaffold; baseline (speedup 1.0000x reference)
import jax
import jax.numpy as jnp
from jax.experimental import pallas as pl


def kernel(x, edge_index, batch, Wl0, bl0, Wr0, Wl1, bl1, Wr1, Wl2, bl2, Wr2, Wm, bm, W1, b1, W2, b2):
    raise NotImplementedError("write your pallas kernel here")


# speedup vs baseline: 9.9470x; 9.9470x over previous
import functools

import jax
import jax.numpy as jnp
from jax import lax
from jax.experimental import pallas as pl
from jax.experimental.pallas import tpu as pltpu
from jax.experimental.pallas import tpu_sc as plsc

_N = 10000
_E = 320000
_DF = 128
_DH = 64
_G = 64
_DT = 10

_NC = 2
_NS = 16
_NW = _NC * _NS

_N_PAD = 10112
_RPT = _N_PAD // _NS
_CH = 128
_EPT = ((_E // _NW + _CH - 1) // _CH) * _CH
_NCH = _EPT // _CH
_E_PAD = _EPT * _NW

_NBLK = 8
_MB = _N_PAD // _NBLK

_mesh = plsc.VectorSubcoreMesh(core_axis_name="c", subcore_axis_name="s")


def _agg_body(table, src, dst, zeros, out, acc, src_v, dst_v, rows_v):
    c = lax.axis_index("c")
    s = lax.axis_index("s")
    wid = s * _NC + c
    pltpu.sync_copy(zeros, acc.at[pl.ds(s * _RPT, _RPT)])
    pltpu.sync_copy(src.at[wid], src_v)
    pltpu.sync_copy(dst.at[wid], dst_v)
    plsc.subcore_barrier()

    def step(j, carry):
        pltpu.sync_copy(table.at[src_v.at[j]], rows_v)
        pltpu.sync_copy(rows_v, acc.at[dst_v.at[j]], add=True)
        return carry

    lax.fori_loop(0, _NCH, step, 0)
    plsc.subcore_barrier()
    pltpu.sync_copy(acc.at[pl.ds(s * _RPT, _RPT)],
                    out.at[c, pl.ds(s * _RPT, _RPT)])


_agg = pl.kernel(
    _agg_body,
    out_type=jax.ShapeDtypeStruct((_NC, _N_PAD, _DH), jnp.float32),
    mesh=_mesh,
    compiler_params=pltpu.CompilerParams(use_tc_tiling_on_sc=False),
    scratch_types=[
        pltpu.VMEM_SHARED((_N_PAD, _DH), jnp.float32),
        pltpu.VMEM((_NCH, _CH), jnp.int32),
        pltpu.VMEM((_NCH, _CH), jnp.int32),
        pltpu.VMEM((_CH, _DH), jnp.float32),
    ],
)


def _cnt_body(dst, ones, zeros, out, acc, dst_v, ones_v):
    c = lax.axis_index("c")
    s = lax.axis_index("s")
    wid = s * _NC + c
    pltpu.sync_copy(zeros, acc.at[pl.ds(s * _RPT, _RPT)])
    pltpu.sync_copy(dst.at[wid], dst_v)
    pltpu.sync_copy(ones, ones_v)
    plsc.subcore_barrier()

    def step(j, carry):
        pltpu.sync_copy(ones_v, acc.at[dst_v.at[j]], add=True)
        return carry

    lax.fori_loop(0, _NCH, step, 0)
    plsc.subcore_barrier()
    pltpu.sync_copy(acc.at[pl.ds(s * _RPT, _RPT)],
                    out.at[c, pl.ds(s * _RPT, _RPT)])


_cnt = pl.kernel(
    _cnt_body,
    out_type=jax.ShapeDtypeStruct((_NC, _N_PAD, 16), jnp.float32),
    mesh=_mesh,
    compiler_params=pltpu.CompilerParams(use_tc_tiling_on_sc=False),
    scratch_types=[
        pltpu.VMEM_SHARED((_N_PAD, 16), jnp.float32),
        pltpu.VMEM((_NCH, _CH), jnp.int32),
        pltpu.VMEM((_CH, 16), jnp.float32),
    ],
)


def _mm2_kernel(h_ref, wa_ref, wb_ref, a_ref, r_ref):
    h = h_ref[...]
    a_ref[...] = jnp.dot(h, wa_ref[...], preferred_element_type=jnp.float32)
    r_ref[...] = jnp.dot(h, wb_ref[...], preferred_element_type=jnp.float32)


def _mm2(h, wa, wb):
    d = h.shape[1]
    return pl.pallas_call(
        _mm2_kernel,
        grid=(_NBLK,),
        in_specs=[
            pl.BlockSpec((_MB, d), lambda i: (i, 0)),
            pl.BlockSpec((d, _DH), lambda i: (0, 0)),
            pl.BlockSpec((d, _DH), lambda i: (0, 0)),
        ],
        out_specs=[
            pl.BlockSpec((_MB, _DH), lambda i: (i, 0)),
            pl.BlockSpec((_MB, _DH), lambda i: (i, 0)),
        ],
        out_shape=[
            jax.ShapeDtypeStruct((_N_PAD, _DH), jnp.float32),
            jax.ShapeDtypeStruct((_N_PAD, _DH), jnp.float32),
        ],
    )(h, wa, wb)


def _comb_kernel(p_ref, cp_ref, r_ref, bl_ref, wm_ref, bm_ref, o_ref):
    p = p_ref[...]
    cp = cp_ref[...]
    cnt = cp[0, :, 0] + cp[1, :, 0]
    rcp = 1.0 / jnp.clip(cnt, 1.0, None)
    pre = (p[0] + p[1]) * rcp[:, None] + bl_ref[...] + r_ref[...]
    z = jnp.dot(pre, wm_ref[...], preferred_element_type=jnp.float32)
    o_ref[...] = jnp.maximum(z + bm_ref[...], 0.0)


def _combine(p, cp, r, bl, wm, bm):
    return pl.pallas_call(
        _comb_kernel,
        grid=(_NBLK,),
        in_specs=[
            pl.BlockSpec((_NC, _MB, _DH), lambda i: (0, i, 0)),
            pl.BlockSpec((_NC, _MB, 16), lambda i: (0, i, 0)),
            pl.BlockSpec((_MB, _DH), lambda i: (i, 0)),
            pl.BlockSpec((1, _DH), lambda i: (0, 0)),
            pl.BlockSpec((_DH, _DH), lambda i: (0, 0)),
            pl.BlockSpec((1, _DH), lambda i: (0, 0)),
        ],
        out_specs=pl.BlockSpec((_MB, _DH), lambda i: (i, 0)),
        out_shape=jax.ShapeDtypeStruct((_N_PAD, _DH), jnp.float32),
    )(p, cp, r, bl, wm, bm)


def _pool_kernel(b_ref, h1_ref, h2_ref, h3_ref, w1_ref, b1_ref, w2_ref,
                 b2_ref, o_ref, acc_ref):
    i = pl.program_id(0)

    @pl.when(i == 0)
    def _():
        acc_ref[...] = jnp.zeros_like(acc_ref)

    bb = b_ref[0, 0, :]
    onehot = (bb[:, None] ==
              lax.broadcasted_iota(jnp.int32, (1, _G), 1)).astype(jnp.float32)
    dn = (((0,), (0,)), ((), ()))
    for k, h_ref in enumerate((h1_ref, h2_ref, h3_ref)):
        acc_ref[:, k * _DH:(k + 1) * _DH] += lax.dot_general(
            onehot, h_ref[...], dn, preferred_element_type=jnp.float32)
    acc_ref[:, 3 * _DH:3 * _DH + 1] += jnp.sum(onehot, axis=0)[:, None]

    @pl.when(i == _NBLK - 1)
    def _():
        accv = acc_ref[...]
        rcp = 1.0 / jnp.clip(accv[:, 3 * _DH:3 * _DH + 1], 1.0, None)
        pooled = accv[:, :3 * _DH] * rcp
        z = jnp.dot(pooled, w1_ref[...], preferred_element_type=jnp.float32)
        z = jnp.maximum(z + b1_ref[...], 0.0)
        o_ref[...] = jnp.dot(
            z, w2_ref[...], preferred_element_type=jnp.float32) + b2_ref[...]


def _pool(batch_r, h1, h2, h3, w1, b1, w2, b2):
    return pl.pallas_call(
        _pool_kernel,
        grid=(_NBLK,),
        in_specs=[
            pl.BlockSpec((1, 1, _MB), lambda i: (i, 0, 0)),
            pl.BlockSpec((_MB, _DH), lambda i: (i, 0)),
            pl.BlockSpec((_MB, _DH), lambda i: (i, 0)),
            pl.BlockSpec((_MB, _DH), lambda i: (i, 0)),
            pl.BlockSpec((3 * _DH, _DH), lambda i: (0, 0)),
            pl.BlockSpec((1, _DH), lambda i: (0, 0)),
            pl.BlockSpec((_DH, _DT), lambda i: (0, 0)),
            pl.BlockSpec((1, _DT), lambda i: (0, 0)),
        ],
        out_specs=pl.BlockSpec((_G, _DT), lambda i: (0, 0)),
        out_shape=jax.ShapeDtypeStruct((_G, _DT), jnp.float32),
        scratch_shapes=[pltpu.VMEM((_G, 3 * _DH + 128), jnp.float32)],
    )(batch_r, h1, h2, h3, w1, b1, w2, b2)


def kernel(x, edge_index, batch, Wl0, bl0, Wr0, Wl1, bl1, Wr1, Wl2, bl2,
           Wr2, Wm, bm, W1, b1, W2, b2):
    x_p = jnp.pad(x, ((0, _N_PAD - _N), (0, 0)))
    src = edge_index[0].astype(jnp.int32)
    dst = edge_index[1].astype(jnp.int32)
    npad = _E_PAD - _E
    ar = jnp.arange(npad, dtype=jnp.int32)
    src_p = jnp.concatenate([src, (ar * 97) % _N]).reshape(_NW, _NCH, _CH)
    dst_p = jnp.concatenate([dst, _N + ar % (_N_PAD - _N)]).reshape(
        _NW, _NCH, _CH)
    zeros64 = jnp.zeros((_RPT, _DH), jnp.float32)
    zeros16 = jnp.zeros((_RPT, 16), jnp.float32)
    ones16 = jnp.ones((_CH, 16), jnp.float32)
    batch_r = jnp.pad(batch.astype(jnp.int32), (0, _N_PAD - _N),
                      constant_values=_G).reshape(_NBLK, 1, _MB)
    blr0, blr1, blr2 = (b.reshape(1, _DH) for b in (bl0, bl1, bl2))
    bmr = bm.reshape(1, _DH)
    b1r = b1.reshape(1, _DH)
    b2r = b2.reshape(1, _DT)

    cp = _cnt(dst_p, ones16, zeros16)

    a0, r0 = _mm2(x_p, Wl0, Wr0)
    p0 = _agg(a0, src_p, dst_p, zeros64)
    h1 = _combine(p0, cp, r0, blr0, Wm, bmr)

    a1, r1 = _mm2(h1, Wl1, Wr1)
    p1 = _agg(a1, src_p, dst_p, zeros64)
    h2 = _combine(p1, cp, r1, blr1, Wm, bmr)

    a2, r2 = _mm2(h2, Wl2, Wr2)
    p2 = _agg(a2, src_p, dst_p, zeros64)
    h3 = _combine(p2, cp, r2, blr2, Wm, bmr)

    return _pool(batch_r, h1, h2, h3, W1, b1r, W2, b2r)


# speedup vs baseline: 13.6856x; 1.3759x over previous
import functools

import jax
import jax.numpy as jnp
from jax import lax
from jax.experimental import pallas as pl
from jax.experimental.pallas import tpu as pltpu
from jax.experimental.pallas import tpu_sc as plsc

_N = 10000
_E = 320000
_DF = 128
_DH = 64
_G = 64
_DT = 10

_NC = 2
_NS = 16
_NW = _NC * _NS

_N_PAD = 10112
_RPT = _N_PAD // _NS
_CH = 128
_EPT = ((_E // _NW + 2 * _CH - 1) // (2 * _CH)) * (2 * _CH)
_NCH = _EPT // _CH
_E_PAD = _EPT * _NW

_NBLK = 8
_MB = _N_PAD // _NBLK

_mesh = plsc.VectorSubcoreMesh(core_axis_name="c", subcore_axis_name="s")


def _agg_body(table, src, dst, zeros, out, acc, src_v, dst_v, rows_a, rows_b,
              gsem_a, gsem_b):
    c = lax.axis_index("c")
    s = lax.axis_index("s")
    wid = s * _NC + c
    pltpu.sync_copy(zeros, acc.at[pl.ds(s * _RPT, _RPT)])
    pltpu.sync_copy(src.at[wid], src_v)
    pltpu.sync_copy(dst.at[wid], dst_v)
    plsc.subcore_barrier()

    pltpu.async_copy(table.at[src_v.at[0]], rows_a, gsem_a)

    def step(i, carry):
        j0 = 2 * i
        j1 = j0 + 1
        pltpu.async_copy(table.at[src_v.at[j1]], rows_b, gsem_b)
        pltpu.make_async_copy(table.at[src_v.at[j0]], rows_a, gsem_a).wait()
        pltpu.sync_copy(rows_a, acc.at[dst_v.at[j0]], add=True)

        @pl.when(j1 + 1 < _NCH)
        def _():
            pltpu.async_copy(table.at[src_v.at[j1 + 1]], rows_a, gsem_a)

        pltpu.make_async_copy(table.at[src_v.at[j1]], rows_b, gsem_b).wait()
        pltpu.sync_copy(rows_b, acc.at[dst_v.at[j1]], add=True)
        return carry

    lax.fori_loop(0, _NCH // 2, step, 0)
    plsc.subcore_barrier()
    pltpu.sync_copy(acc.at[pl.ds(s * _RPT, _RPT)],
                    out.at[c, pl.ds(s * _RPT, _RPT)])


_agg = pl.kernel(
    _agg_body,
    out_type=jax.ShapeDtypeStruct((_NC, _N_PAD, _DH), jnp.float32),
    mesh=_mesh,
    compiler_params=pltpu.CompilerParams(use_tc_tiling_on_sc=False),
    scratch_types=[
        pltpu.VMEM_SHARED((_N_PAD, _DH), jnp.float32),
        pltpu.VMEM((_NCH, _CH), jnp.int32),
        pltpu.VMEM((_NCH, _CH), jnp.int32),
        pltpu.VMEM((_CH, _DH), jnp.float32),
        pltpu.VMEM((_CH, _DH), jnp.float32),
        pltpu.SemaphoreType.DMA,
        pltpu.SemaphoreType.DMA,
    ],
)


def _cnt_body(dst, ones, zeros, out, acc, dst_v, ones_v):
    c = lax.axis_index("c")
    s = lax.axis_index("s")
    wid = s * _NC + c
    pltpu.sync_copy(zeros, acc.at[pl.ds(s * _RPT, _RPT)])
    pltpu.sync_copy(dst.at[wid], dst_v)
    pltpu.sync_copy(ones, ones_v)
    plsc.subcore_barrier()

    def step(j, carry):
        pltpu.sync_copy(ones_v, acc.at[dst_v.at[j]], add=True)
        return carry

    lax.fori_loop(0, _NCH, step, 0)
    plsc.subcore_barrier()
    pltpu.sync_copy(acc.at[pl.ds(s * _RPT, _RPT)],
                    out.at[c, pl.ds(s * _RPT, _RPT)])


_cnt = pl.kernel(
    _cnt_body,
    out_type=jax.ShapeDtypeStruct((_NC, _N_PAD, 16), jnp.float32),
    mesh=_mesh,
    compiler_params=pltpu.CompilerParams(use_tc_tiling_on_sc=False),
    scratch_types=[
        pltpu.VMEM_SHARED((_N_PAD, 16), jnp.float32),
        pltpu.VMEM((_NCH, _CH), jnp.int32),
        pltpu.VMEM((_CH, 16), jnp.float32),
    ],
)


def _mm2_kernel(h_ref, wa_ref, wb_ref, a_ref, r_ref):
    h = h_ref[...]
    a_ref[...] = jnp.dot(h, wa_ref[...], preferred_element_type=jnp.float32)
    r_ref[...] = jnp.dot(h, wb_ref[...], preferred_element_type=jnp.float32)


def _mm2(h, wa, wb):
    d = h.shape[1]
    return pl.pallas_call(
        _mm2_kernel,
        grid=(_NBLK,),
        in_specs=[
            pl.BlockSpec((_MB, d), lambda i: (i, 0)),
            pl.BlockSpec((d, _DH), lambda i: (0, 0)),
            pl.BlockSpec((d, _DH), lambda i: (0, 0)),
        ],
        out_specs=[
            pl.BlockSpec((_MB, _DH), lambda i: (i, 0)),
            pl.BlockSpec((_MB, _DH), lambda i: (i, 0)),
        ],
        out_shape=[
            jax.ShapeDtypeStruct((_N_PAD, _DH), jnp.float32),
            jax.ShapeDtypeStruct((_N_PAD, _DH), jnp.float32),
        ],
    )(h, wa, wb)


def _comb_kernel(p_ref, cp_ref, r_ref, bl_ref, wm_ref, bm_ref, o_ref):
    p = p_ref[...]
    cp = cp_ref[...]
    cnt = cp[0, :, 0] + cp[1, :, 0]
    rcp = 1.0 / jnp.clip(cnt, 1.0, None)
    pre = (p[0] + p[1]) * rcp[:, None] + bl_ref[...] + r_ref[...]
    z = jnp.dot(pre, wm_ref[...], preferred_element_type=jnp.float32)
    o_ref[...] = jnp.maximum(z + bm_ref[...], 0.0)


def _combine(p, cp, r, bl, wm, bm):
    return pl.pallas_call(
        _comb_kernel,
        grid=(_NBLK,),
        in_specs=[
            pl.BlockSpec((_NC, _MB, _DH), lambda i: (0, i, 0)),
            pl.BlockSpec((_NC, _MB, 16), lambda i: (0, i, 0)),
            pl.BlockSpec((_MB, _DH), lambda i: (i, 0)),
            pl.BlockSpec((1, _DH), lambda i: (0, 0)),
            pl.BlockSpec((_DH, _DH), lambda i: (0, 0)),
            pl.BlockSpec((1, _DH), lambda i: (0, 0)),
        ],
        out_specs=pl.BlockSpec((_MB, _DH), lambda i: (i, 0)),
        out_shape=jax.ShapeDtypeStruct((_N_PAD, _DH), jnp.float32),
    )(p, cp, r, bl, wm, bm)


def _pool_kernel(b_ref, h1_ref, h2_ref, h3_ref, w1_ref, b1_ref, w2_ref,
                 b2_ref, o_ref, acc_ref):
    i = pl.program_id(0)

    @pl.when(i == 0)
    def _():
        acc_ref[...] = jnp.zeros_like(acc_ref)

    bb = b_ref[0, 0, :]
    onehot = (bb[:, None] ==
              lax.broadcasted_iota(jnp.int32, (1, _G), 1)).astype(jnp.float32)
    dn = (((0,), (0,)), ((), ()))
    for k, h_ref in enumerate((h1_ref, h2_ref, h3_ref)):
        acc_ref[:, k * _DH:(k + 1) * _DH] += lax.dot_general(
            onehot, h_ref[...], dn, preferred_element_type=jnp.float32)
    acc_ref[:, 3 * _DH:3 * _DH + 1] += jnp.sum(onehot, axis=0)[:, None]

    @pl.when(i == _NBLK - 1)
    def _():
        accv = acc_ref[...]
        rcp = 1.0 / jnp.clip(accv[:, 3 * _DH:3 * _DH + 1], 1.0, None)
        pooled = accv[:, :3 * _DH] * rcp
        z = jnp.dot(pooled, w1_ref[...], preferred_element_type=jnp.float32)
        z = jnp.maximum(z + b1_ref[...], 0.0)
        o_ref[...] = jnp.dot(
            z, w2_ref[...], preferred_element_type=jnp.float32) + b2_ref[...]


def _pool(batch_r, h1, h2, h3, w1, b1, w2, b2):
    return pl.pallas_call(
        _pool_kernel,
        grid=(_NBLK,),
        in_specs=[
            pl.BlockSpec((1, 1, _MB), lambda i: (i, 0, 0)),
            pl.BlockSpec((_MB, _DH), lambda i: (i, 0)),
            pl.BlockSpec((_MB, _DH), lambda i: (i, 0)),
            pl.BlockSpec((_MB, _DH), lambda i: (i, 0)),
            pl.BlockSpec((3 * _DH, _DH), lambda i: (0, 0)),
            pl.BlockSpec((1, _DH), lambda i: (0, 0)),
            pl.BlockSpec((_DH, _DT), lambda i: (0, 0)),
            pl.BlockSpec((1, _DT), lambda i: (0, 0)),
        ],
        out_specs=pl.BlockSpec((_G, _DT), lambda i: (0, 0)),
        out_shape=jax.ShapeDtypeStruct((_G, _DT), jnp.float32),
        scratch_shapes=[pltpu.VMEM((_G, 3 * _DH + 128), jnp.float32)],
    )(batch_r, h1, h2, h3, w1, b1, w2, b2)


def kernel(x, edge_index, batch, Wl0, bl0, Wr0, Wl1, bl1, Wr1, Wl2, bl2,
           Wr2, Wm, bm, W1, b1, W2, b2):
    x_p = jnp.pad(x, ((0, _N_PAD - _N), (0, 0)))
    src = edge_index[0].astype(jnp.int32)
    dst = edge_index[1].astype(jnp.int32)
    npad = _E_PAD - _E
    ar = jnp.arange(npad, dtype=jnp.int32)
    src_p = jnp.concatenate([src, (ar * 97) % _N]).reshape(_NW, _NCH, _CH)
    dst_p = jnp.concatenate([dst, _N + ar % (_N_PAD - _N)]).reshape(
        _NW, _NCH, _CH)
    zeros64 = jnp.zeros((_RPT, _DH), jnp.float32)
    zeros16 = jnp.zeros((_RPT, 16), jnp.float32)
    ones16 = jnp.ones((_CH, 16), jnp.float32)
    batch_r = jnp.pad(batch.astype(jnp.int32), (0, _N_PAD - _N),
                      constant_values=_G).reshape(_NBLK, 1, _MB)
    blr0, blr1, blr2 = (b.reshape(1, _DH) for b in (bl0, bl1, bl2))
    bmr = bm.reshape(1, _DH)
    b1r = b1.reshape(1, _DH)
    b2r = b2.reshape(1, _DT)

    cp = _cnt(dst_p, ones16, zeros16)

    a0, r0 = _mm2(x_p, Wl0, Wr0)
    p0 = _agg(a0, src_p, dst_p, zeros64)
    h1 = _combine(p0, cp, r0, blr0, Wm, bmr)

    a1, r1 = _mm2(h1, Wl1, Wr1)
    p1 = _agg(a1, src_p, dst_p, zeros64)
    h2 = _combine(p1, cp, r1, blr1, Wm, bmr)

    a2, r2 = _mm2(h2, Wl2, Wr2)
    p2 = _agg(a2, src_p, dst_p, zeros64)
    h3 = _combine(p2, cp, r2, blr2, Wm, bmr)

    return _pool(batch_r, h1, h2, h3, W1, b1r, W2, b2r)


# speedup vs baseline: 14.3294x; 1.0470x over previous
import functools

import jax
import jax.numpy as jnp
from jax import lax
from jax.experimental import pallas as pl
from jax.experimental.pallas import tpu as pltpu
from jax.experimental.pallas import tpu_sc as plsc

_N = 10000
_E = 320000
_DF = 128
_DH = 64
_G = 64
_DT = 10

_NC = 2
_NS = 16
_NW = _NC * _NS

_N_PAD = 10112
_RPT = _N_PAD // _NS
_CH = 128
_EPT = ((_E // _NW + 2 * _CH - 1) // (2 * _CH)) * (2 * _CH)
_NCH = _EPT // _CH
_E_PAD = _EPT * _NW

_NBLK = 8
_MB = _N_PAD // _NBLK

_mesh = plsc.VectorSubcoreMesh(core_axis_name="c", subcore_axis_name="s")


def _agg_body(table, src, dst, zeros, out, acc, src_v, dst_v, rows_a, rows_b,
              gsem_a, gsem_b):
    c = lax.axis_index("c")
    s = lax.axis_index("s")
    wid = s * _NC + c
    pltpu.sync_copy(zeros, acc.at[pl.ds(s * _RPT, _RPT)])
    pltpu.sync_copy(src.at[wid], src_v)
    pltpu.sync_copy(dst.at[wid], dst_v)
    plsc.subcore_barrier()

    pltpu.async_copy(table.at[src_v.at[0]], rows_a, gsem_a)

    def step(i, carry):
        j0 = 2 * i
        j1 = j0 + 1
        pltpu.async_copy(table.at[src_v.at[j1]], rows_b, gsem_b)
        pltpu.make_async_copy(table.at[src_v.at[j0]], rows_a, gsem_a).wait()
        pltpu.sync_copy(rows_a, acc.at[dst_v.at[j0]], add=True)

        @pl.when(j1 + 1 < _NCH)
        def _():
            pltpu.async_copy(table.at[src_v.at[j1 + 1]], rows_a, gsem_a)

        pltpu.make_async_copy(table.at[src_v.at[j1]], rows_b, gsem_b).wait()
        pltpu.sync_copy(rows_b, acc.at[dst_v.at[j1]], add=True)
        return carry

    lax.fori_loop(0, _NCH // 2, step, 0)
    plsc.subcore_barrier()
    pltpu.sync_copy(acc.at[pl.ds(s * _RPT, _RPT)],
                    out.at[c, pl.ds(s * _RPT, _RPT)])


_agg = pl.kernel(
    _agg_body,
    out_type=jax.ShapeDtypeStruct((_NC, _N_PAD, _DH), jnp.float32),
    mesh=_mesh,
    compiler_params=pltpu.CompilerParams(use_tc_tiling_on_sc=False),
    scratch_types=[
        pltpu.VMEM_SHARED((_N_PAD, _DH), jnp.float32),
        pltpu.VMEM((_NCH, _CH), jnp.int32),
        pltpu.VMEM((_NCH, _CH), jnp.int32),
        pltpu.VMEM((_CH, _DH), jnp.float32),
        pltpu.VMEM((_CH, _DH), jnp.float32),
        pltpu.SemaphoreType.DMA,
        pltpu.SemaphoreType.DMA,
    ],
)


def _cnt_body(dst, ones, zeros, out, acc, dst_v, ones_v):
    c = lax.axis_index("c")
    s = lax.axis_index("s")
    wid = s * _NC + c
    pltpu.sync_copy(zeros, acc.at[pl.ds(s * _RPT, _RPT)])
    pltpu.sync_copy(dst.at[wid], dst_v)
    pltpu.sync_copy(ones, ones_v)
    plsc.subcore_barrier()

    def step(j, carry):
        pltpu.sync_copy(ones_v, acc.at[dst_v.at[j]], add=True)
        return carry

    lax.fori_loop(0, _NCH, step, 0)
    plsc.subcore_barrier()
    pltpu.sync_copy(acc.at[pl.ds(s * _RPT, _RPT)],
                    out.at[c, pl.ds(s * _RPT, _RPT)])


_cnt = pl.kernel(
    _cnt_body,
    out_type=jax.ShapeDtypeStruct((_NC, _N_PAD, 16), jnp.float32),
    mesh=_mesh,
    compiler_params=pltpu.CompilerParams(use_tc_tiling_on_sc=False),
    scratch_types=[
        pltpu.VMEM_SHARED((_N_PAD, 16), jnp.float32),
        pltpu.VMEM((_NCH, _CH), jnp.int32),
        pltpu.VMEM((_CH, 16), jnp.float32),
    ],
)


def _wfold_kernel(wl0_ref, wr0_ref, wl1_ref, wr1_ref, wl2_ref, wr2_ref,
                  wm_ref, bl0_ref, bl1_ref, bl2_ref, bm_ref, *out_refs):
    wm = wm_ref[...]
    for k, w_ref in enumerate((wl0_ref, wr0_ref, wl1_ref, wr1_ref, wl2_ref,
                               wr2_ref)):
        out_refs[k][...] = jnp.dot(w_ref[...], wm,
                                   preferred_element_type=jnp.float32)
    bm = bm_ref[...]
    for k, b_ref in enumerate((bl0_ref, bl1_ref, bl2_ref)):
        out_refs[6 + k][...] = jnp.dot(b_ref[...], wm,
                                       preferred_element_type=jnp.float32) + bm


def _wfold(wl0, wr0, wl1, wr1, wl2, wr2, wm, bl0, bl1, bl2, bm):
    full = lambda shape: pl.BlockSpec(shape, lambda: (0, 0))
    return pl.pallas_call(
        _wfold_kernel,
        in_specs=[full((_DF, _DH)), full((_DF, _DH))] +
                 [full((_DH, _DH))] * 5 + [full((1, _DH))] * 4,
        out_specs=[full((_DF, _DH)), full((_DF, _DH))] +
                  [full((_DH, _DH))] * 4 + [full((1, _DH))] * 3,
        out_shape=[jax.ShapeDtypeStruct((_DF, _DH), jnp.float32)] * 2 +
                  [jax.ShapeDtypeStruct((_DH, _DH), jnp.float32)] * 4 +
                  [jax.ShapeDtypeStruct((1, _DH), jnp.float32)] * 3,
    )(wl0, wr0, wl1, wr1, wl2, wr2, wm, bl0, bl1, bl2, bm)


def _mm2_kernel(h_ref, wa_ref, wb_ref, a_ref, r_ref):
    h = h_ref[...]
    a_ref[...] = jnp.dot(h, wa_ref[...], preferred_element_type=jnp.float32)
    r_ref[...] = jnp.dot(h, wb_ref[...], preferred_element_type=jnp.float32)


def _mm2(h, wa, wb):
    d = h.shape[1]
    return pl.pallas_call(
        _mm2_kernel,
        grid=(_NBLK,),
        in_specs=[
            pl.BlockSpec((_MB, d), lambda i: (i, 0)),
            pl.BlockSpec((d, _DH), lambda i: (0, 0)),
            pl.BlockSpec((d, _DH), lambda i: (0, 0)),
        ],
        out_specs=[
            pl.BlockSpec((_MB, _DH), lambda i: (i, 0)),
            pl.BlockSpec((_MB, _DH), lambda i: (i, 0)),
        ],
        out_shape=[
            jax.ShapeDtypeStruct((_N_PAD, _DH), jnp.float32),
            jax.ShapeDtypeStruct((_N_PAD, _DH), jnp.float32),
        ],
    )(h, wa, wb)


def _hcomb(p, cp, r, bb):
    cnt = cp[0, :, 0] + cp[1, :, 0]
    rcp = 1.0 / jnp.clip(cnt, 1.0, None)
    return jnp.maximum((p[0] + p[1]) * rcp[:, None] + bb + r, 0.0)


def _cmb2_kernel(p_ref, cp_ref, r_ref, bb_ref, wa_ref, wb_ref, h_ref, a_ref,
                 r2_ref):
    h = _hcomb(p_ref[...], cp_ref[...], r_ref[...], bb_ref[...])
    h_ref[...] = h
    a_ref[...] = jnp.dot(h, wa_ref[...], preferred_element_type=jnp.float32)
    r2_ref[...] = jnp.dot(h, wb_ref[...], preferred_element_type=jnp.float32)


def _cmb2(p, cp, r, bb, wa, wb):
    return pl.pallas_call(
        _cmb2_kernel,
        grid=(_NBLK,),
        in_specs=[
            pl.BlockSpec((_NC, _MB, _DH), lambda i: (0, i, 0)),
            pl.BlockSpec((_NC, _MB, 16), lambda i: (0, i, 0)),
            pl.BlockSpec((_MB, _DH), lambda i: (i, 0)),
            pl.BlockSpec((1, _DH), lambda i: (0, 0)),
            pl.BlockSpec((_DH, _DH), lambda i: (0, 0)),
            pl.BlockSpec((_DH, _DH), lambda i: (0, 0)),
        ],
        out_specs=[
            pl.BlockSpec((_MB, _DH), lambda i: (i, 0)),
            pl.BlockSpec((_MB, _DH), lambda i: (i, 0)),
            pl.BlockSpec((_MB, _DH), lambda i: (i, 0)),
        ],
        out_shape=[jax.ShapeDtypeStruct((_N_PAD, _DH), jnp.float32)] * 3,
    )(p, cp, r, bb, wa, wb)


def _pool_kernel(b_ref, h1_ref, h2_ref, p_ref, cp_ref, r_ref, bb_ref,
                 w1_ref, b1_ref, w2_ref, b2_ref, o_ref, acc_ref):
    i = pl.program_id(0)

    @pl.when(i == 0)
    def _():
        acc_ref[...] = jnp.zeros_like(acc_ref)

    h3 = _hcomb(p_ref[...], cp_ref[...], r_ref[...], bb_ref[...])
    bb = b_ref[0, 0, :]
    onehot = (bb[:, None] ==
              lax.broadcasted_iota(jnp.int32, (1, _G), 1)).astype(jnp.float32)
    dn = (((0,), (0,)), ((), ()))
    for k, h in enumerate((h1_ref[...], h2_ref[...], h3)):
        acc_ref[:, k * _DH:(k + 1) * _DH] += lax.dot_general(
            onehot, h, dn, preferred_element_type=jnp.float32)
    acc_ref[:, 3 * _DH:3 * _DH + 1] += jnp.sum(onehot, axis=0)[:, None]

    @pl.when(i == _NBLK - 1)
    def _():
        accv = acc_ref[...]
        rcp = 1.0 / jnp.clip(accv[:, 3 * _DH:3 * _DH + 1], 1.0, None)
        pooled = accv[:, :3 * _DH] * rcp
        z = jnp.dot(pooled, w1_ref[...], preferred_element_type=jnp.float32)
        z = jnp.maximum(z + b1_ref[...], 0.0)
        o_ref[...] = jnp.dot(
            z, w2_ref[...], preferred_element_type=jnp.float32) + b2_ref[...]


def _pool(batch_r, h1, h2, p, cp, r, bb, w1, b1, w2, b2):
    return pl.pallas_call(
        _pool_kernel,
        grid=(_NBLK,),
        in_specs=[
            pl.BlockSpec((1, 1, _MB), lambda i: (i, 0, 0)),
            pl.BlockSpec((_MB, _DH), lambda i: (i, 0)),
            pl.BlockSpec((_MB, _DH), lambda i: (i, 0)),
            pl.BlockSpec((_NC, _MB, _DH), lambda i: (0, i, 0)),
            pl.BlockSpec((_NC, _MB, 16), lambda i: (0, i, 0)),
            pl.BlockSpec((_MB, _DH), lambda i: (i, 0)),
            pl.BlockSpec((1, _DH), lambda i: (0, 0)),
            pl.BlockSpec((3 * _DH, _DH), lambda i: (0, 0)),
            pl.BlockSpec((1, _DH), lambda i: (0, 0)),
            pl.BlockSpec((_DH, _DT), lambda i: (0, 0)),
            pl.BlockSpec((1, _DT), lambda i: (0, 0)),
        ],
        out_specs=pl.BlockSpec((_G, _DT), lambda i: (0, 0)),
        out_shape=jax.ShapeDtypeStruct((_G, _DT), jnp.float32),
        scratch_shapes=[pltpu.VMEM((_G, 3 * _DH + 128), jnp.float32)],
    )(batch_r, h1, h2, p, cp, r, bb, w1, b1, w2, b2)


def kernel(x, edge_index, batch, Wl0, bl0, Wr0, Wl1, bl1, Wr1, Wl2, bl2,
           Wr2, Wm, bm, W1, b1, W2, b2):
    x_p = jnp.pad(x, ((0, _N_PAD - _N), (0, 0)))
    src = edge_index[0].astype(jnp.int32)
    dst = edge_index[1].astype(jnp.int32)
    npad = _E_PAD - _E
    ar = jnp.arange(npad, dtype=jnp.int32)
    src_p = jnp.concatenate([src, (ar * 97) % _N]).reshape(_NW, _NCH, _CH)
    dst_p = jnp.concatenate([dst, _N + ar % (_N_PAD - _N)]).reshape(
        _NW, _NCH, _CH)
    zeros64 = jnp.zeros((_RPT, _DH), jnp.float32)
    zeros16 = jnp.zeros((_RPT, 16), jnp.float32)
    ones16 = jnp.ones((_CH, 16), jnp.float32)
    batch_r = jnp.pad(batch.astype(jnp.int32), (0, _N_PAD - _N),
                      constant_values=_G).reshape(_NBLK, 1, _MB)
    blr0, blr1, blr2 = (b.reshape(1, _DH) for b in (bl0, bl1, bl2))
    bmr = bm.reshape(1, _DH)
    b1r = b1.reshape(1, _DH)
    b2r = b2.reshape(1, _DT)

    (wlm0, wrm0, wlm1, wrm1, wlm2, wrm2, bb0, bb1, bb2) = _wfold(
        Wl0, Wr0, Wl1, Wr1, Wl2, Wr2, Wm, blr0, blr1, blr2, bmr)

    cp = _cnt(dst_p, ones16, zeros16)

    a0, r0 = _mm2(x_p, wlm0, wrm0)
    p0 = _agg(a0, src_p, dst_p, zeros64)
    h1, a1, r1 = _cmb2(p0, cp, r0, bb0, wlm1, wrm1)

    p1 = _agg(a1, src_p, dst_p, zeros64)
    h2, a2, r2 = _cmb2(p1, cp, r1, bb1, wlm2, wrm2)

    p2 = _agg(a2, src_p, dst_p, zeros64)

    return _pool(batch_r, h1, h2, p2, cp, r2, bb2, W1, b1r, W2, b2r)


# speedup vs baseline: 16.0035x; 1.1168x over previous
import functools

import jax
import jax.numpy as jnp
from jax import lax
from jax.experimental import pallas as pl
from jax.experimental.pallas import tpu as pltpu
from jax.experimental.pallas import tpu_sc as plsc

_N = 10000
_E = 320000
_DF = 128
_DH = 64
_G = 64
_DT = 10

_NC = 2
_NS = 16
_NW = _NC * _NS

_N_PAD = 10112
_RPT = _N_PAD // _NS
_CH = 128
_EPT = ((_E // _NW + 2 * _CH - 1) // (2 * _CH)) * (2 * _CH)
_NCH = _EPT // _CH
_E_PAD = _EPT * _NW

_NBLK = 8
_MB = _N_PAD // _NBLK

_mesh = plsc.VectorSubcoreMesh(core_axis_name="c", subcore_axis_name="s")


def _agg_body(table, src, dst, zeros, out, acc, src_v, dst_v, rows, gsem,
              ssem):
    c = lax.axis_index("c")
    s = lax.axis_index("s")
    wid = s * _NC + c
    pltpu.sync_copy(zeros, acc.at[pl.ds(s * _RPT, _RPT)])
    pltpu.sync_copy(src.at[wid], src_v)
    pltpu.sync_copy(dst.at[wid], dst_v)
    plsc.subcore_barrier()

    for k in range(_NIF):
        pltpu.async_copy(table.at[src_v.at[k]], rows[k], gsem[k])

    def step(i, carry):
        base = i * _NBUF
        for k in range(_NBUF):
            j = base + k
            jn = j + _NIF
            bn = (k + _NIF) % _NBUF

            @pl.when(jnp.logical_and(jn < _NCH, j >= _NBUF - _NIF))
            def _():
                pltpu.make_async_copy(rows[bn], acc.at[dst_v.at[jn - _NBUF]],
                                      ssem[bn]).wait()

            @pl.when(jn < _NCH)
            def _():
                pltpu.async_copy(table.at[src_v.at[jn]], rows[bn], gsem[bn])

            pltpu.make_async_copy(table.at[src_v.at[j]], rows[k],
                                  gsem[k]).wait()
            pltpu.async_copy(rows[k], acc.at[dst_v.at[j]], ssem[k], add=True)
        return carry

    lax.fori_loop(0, _NCH // _NBUF, step, 0)
    for k in range(_NBUF):
        pltpu.make_async_copy(rows[k], acc.at[dst_v.at[_NCH - _NBUF + k]],
                              ssem[k]).wait()
    plsc.subcore_barrier()
    pltpu.sync_copy(acc.at[pl.ds(s * _RPT, _RPT)],
                    out.at[c, pl.ds(s * _RPT, _RPT)])


_NBUF = 8
_NIF = 4
assert _NCH % _NBUF == 0 and _NIF <= _NBUF

_agg = pl.kernel(
    lambda table, src, dst, zeros, out, acc, src_v, dst_v, *bufs: _agg_body(
        table, src, dst, zeros, out, acc, src_v, dst_v,
        list(bufs[:_NBUF]), list(bufs[_NBUF:2 * _NBUF]),
        list(bufs[2 * _NBUF:])),
    out_type=jax.ShapeDtypeStruct((_NC, _N_PAD, _DH), jnp.float32),
    mesh=_mesh,
    compiler_params=pltpu.CompilerParams(use_tc_tiling_on_sc=False),
    scratch_types=[
        pltpu.VMEM_SHARED((_N_PAD, _DH), jnp.float32),
        pltpu.VMEM((_NCH, _CH), jnp.int32),
        pltpu.VMEM((_NCH, _CH), jnp.int32),
    ] + [pltpu.VMEM((_CH, _DH), jnp.float32)] * _NBUF
      + [pltpu.SemaphoreType.DMA] * (2 * _NBUF),
)


def _cnt_body(dst, ones, zeros, out, acc, dst_v, ones_v):
    c = lax.axis_index("c")
    s = lax.axis_index("s")
    wid = s * _NC + c
    pltpu.sync_copy(zeros, acc.at[pl.ds(s * _RPT, _RPT)])
    pltpu.sync_copy(dst.at[wid], dst_v)
    pltpu.sync_copy(ones, ones_v)
    plsc.subcore_barrier()

    def step(j, carry):
        pltpu.sync_copy(ones_v, acc.at[dst_v.at[j]], add=True)
        return carry

    lax.fori_loop(0, _NCH, step, 0)
    plsc.subcore_barrier()
    pltpu.sync_copy(acc.at[pl.ds(s * _RPT, _RPT)],
                    out.at[c, pl.ds(s * _RPT, _RPT)])


_cnt = pl.kernel(
    _cnt_body,
    out_type=jax.ShapeDtypeStruct((_NC, _N_PAD, 16), jnp.float32),
    mesh=_mesh,
    compiler_params=pltpu.CompilerParams(use_tc_tiling_on_sc=False),
    scratch_types=[
        pltpu.VMEM_SHARED((_N_PAD, 16), jnp.float32),
        pltpu.VMEM((_NCH, _CH), jnp.int32),
        pltpu.VMEM((_CH, 16), jnp.float32),
    ],
)


def _wfold_kernel(wl0_ref, wr0_ref, wl1_ref, wr1_ref, wl2_ref, wr2_ref,
                  wm_ref, bl0_ref, bl1_ref, bl2_ref, bm_ref, *out_refs):
    wm = wm_ref[...]
    for k, w_ref in enumerate((wl0_ref, wr0_ref, wl1_ref, wr1_ref, wl2_ref,
                               wr2_ref)):
        out_refs[k][...] = jnp.dot(w_ref[...], wm,
                                   preferred_element_type=jnp.float32)
    bm = bm_ref[...]
    for k, b_ref in enumerate((bl0_ref, bl1_ref, bl2_ref)):
        out_refs[6 + k][...] = jnp.dot(b_ref[...], wm,
                                       preferred_element_type=jnp.float32) + bm


def _wfold(wl0, wr0, wl1, wr1, wl2, wr2, wm, bl0, bl1, bl2, bm):
    full = lambda shape: pl.BlockSpec(shape, lambda: (0, 0))
    return pl.pallas_call(
        _wfold_kernel,
        in_specs=[full((_DF, _DH)), full((_DF, _DH))] +
                 [full((_DH, _DH))] * 5 + [full((1, _DH))] * 4,
        out_specs=[full((_DF, _DH)), full((_DF, _DH))] +
                  [full((_DH, _DH))] * 4 + [full((1, _DH))] * 3,
        out_shape=[jax.ShapeDtypeStruct((_DF, _DH), jnp.float32)] * 2 +
                  [jax.ShapeDtypeStruct((_DH, _DH), jnp.float32)] * 4 +
                  [jax.ShapeDtypeStruct((1, _DH), jnp.float32)] * 3,
    )(wl0, wr0, wl1, wr1, wl2, wr2, wm, bl0, bl1, bl2, bm)


def _mm2_kernel(h_ref, wa_ref, wb_ref, a_ref, r_ref):
    h = h_ref[...]
    a_ref[...] = jnp.dot(h, wa_ref[...], preferred_element_type=jnp.float32)
    r_ref[...] = jnp.dot(h, wb_ref[...], preferred_element_type=jnp.float32)


def _mm2(h, wa, wb):
    d = h.shape[1]
    return pl.pallas_call(
        _mm2_kernel,
        grid=(_NBLK,),
        in_specs=[
            pl.BlockSpec((_MB, d), lambda i: (i, 0)),
            pl.BlockSpec((d, _DH), lambda i: (0, 0)),
            pl.BlockSpec((d, _DH), lambda i: (0, 0)),
        ],
        out_specs=[
            pl.BlockSpec((_MB, _DH), lambda i: (i, 0)),
            pl.BlockSpec((_MB, _DH), lambda i: (i, 0)),
        ],
        out_shape=[
            jax.ShapeDtypeStruct((_N_PAD, _DH), jnp.float32),
            jax.ShapeDtypeStruct((_N_PAD, _DH), jnp.float32),
        ],
    )(h, wa, wb)


def _hcomb(p, cp, r, bb):
    cnt = cp[0, :, 0] + cp[1, :, 0]
    rcp = 1.0 / jnp.clip(cnt, 1.0, None)
    return jnp.maximum((p[0] + p[1]) * rcp[:, None] + bb + r, 0.0)


def _cmb2_kernel(p_ref, cp_ref, r_ref, bb_ref, wa_ref, wb_ref, h_ref, a_ref,
                 r2_ref):
    h = _hcomb(p_ref[...], cp_ref[...], r_ref[...], bb_ref[...])
    h_ref[...] = h
    a_ref[...] = jnp.dot(h, wa_ref[...], preferred_element_type=jnp.float32)
    r2_ref[...] = jnp.dot(h, wb_ref[...], preferred_element_type=jnp.float32)


def _cmb2(p, cp, r, bb, wa, wb):
    return pl.pallas_call(
        _cmb2_kernel,
        grid=(_NBLK,),
        in_specs=[
            pl.BlockSpec((_NC, _MB, _DH), lambda i: (0, i, 0)),
            pl.BlockSpec((_NC, _MB, 16), lambda i: (0, i, 0)),
            pl.BlockSpec((_MB, _DH), lambda i: (i, 0)),
            pl.BlockSpec((1, _DH), lambda i: (0, 0)),
            pl.BlockSpec((_DH, _DH), lambda i: (0, 0)),
            pl.BlockSpec((_DH, _DH), lambda i: (0, 0)),
        ],
        out_specs=[
            pl.BlockSpec((_MB, _DH), lambda i: (i, 0)),
            pl.BlockSpec((_MB, _DH), lambda i: (i, 0)),
            pl.BlockSpec((_MB, _DH), lambda i: (i, 0)),
        ],
        out_shape=[jax.ShapeDtypeStruct((_N_PAD, _DH), jnp.float32)] * 3,
    )(p, cp, r, bb, wa, wb)


def _pool_kernel(b_ref, h1_ref, h2_ref, p_ref, cp_ref, r_ref, bb_ref,
                 w1_ref, b1_ref, w2_ref, b2_ref, o_ref, acc_ref):
    i = pl.program_id(0)

    @pl.when(i == 0)
    def _():
        acc_ref[...] = jnp.zeros_like(acc_ref)

    h3 = _hcomb(p_ref[...], cp_ref[...], r_ref[...], bb_ref[...])
    bb = b_ref[0, 0, :]
    onehot = (bb[:, None] ==
              lax.broadcasted_iota(jnp.int32, (1, _G), 1)).astype(jnp.float32)
    dn = (((0,), (0,)), ((), ()))
    for k, h in enumerate((h1_ref[...], h2_ref[...], h3)):
        acc_ref[:, k * _DH:(k + 1) * _DH] += lax.dot_general(
            onehot, h, dn, preferred_element_type=jnp.float32)
    acc_ref[:, 3 * _DH:3 * _DH + 1] += jnp.sum(onehot, axis=0)[:, None]

    @pl.when(i == _NBLK - 1)
    def _():
        accv = acc_ref[...]
        rcp = 1.0 / jnp.clip(accv[:, 3 * _DH:3 * _DH + 1], 1.0, None)
        pooled = accv[:, :3 * _DH] * rcp
        z = jnp.dot(pooled, w1_ref[...], preferred_element_type=jnp.float32)
        z = jnp.maximum(z + b1_ref[...], 0.0)
        o_ref[...] = jnp.dot(
            z, w2_ref[...], preferred_element_type=jnp.float32) + b2_ref[...]


def _pool(batch_r, h1, h2, p, cp, r, bb, w1, b1, w2, b2):
    return pl.pallas_call(
        _pool_kernel,
        grid=(_NBLK,),
        in_specs=[
            pl.BlockSpec((1, 1, _MB), lambda i: (i, 0, 0)),
            pl.BlockSpec((_MB, _DH), lambda i: (i, 0)),
            pl.BlockSpec((_MB, _DH), lambda i: (i, 0)),
            pl.BlockSpec((_NC, _MB, _DH), lambda i: (0, i, 0)),
            pl.BlockSpec((_NC, _MB, 16), lambda i: (0, i, 0)),
            pl.BlockSpec((_MB, _DH), lambda i: (i, 0)),
            pl.BlockSpec((1, _DH), lambda i: (0, 0)),
            pl.BlockSpec((3 * _DH, _DH), lambda i: (0, 0)),
            pl.BlockSpec((1, _DH), lambda i: (0, 0)),
            pl.BlockSpec((_DH, _DT), lambda i: (0, 0)),
            pl.BlockSpec((1, _DT), lambda i: (0, 0)),
        ],
        out_specs=pl.BlockSpec((_G, _DT), lambda i: (0, 0)),
        out_shape=jax.ShapeDtypeStruct((_G, _DT), jnp.float32),
        scratch_shapes=[pltpu.VMEM((_G, 3 * _DH + 128), jnp.float32)],
    )(batch_r, h1, h2, p, cp, r, bb, w1, b1, w2, b2)


def kernel(x, edge_index, batch, Wl0, bl0, Wr0, Wl1, bl1, Wr1, Wl2, bl2,
           Wr2, Wm, bm, W1, b1, W2, b2):
    x_p = jnp.pad(x, ((0, _N_PAD - _N), (0, 0)))
    src = edge_index[0].astype(jnp.int32)
    dst = edge_index[1].astype(jnp.int32)
    npad = _E_PAD - _E
    ar = jnp.arange(npad, dtype=jnp.int32)
    src_p = jnp.concatenate([src, (ar * 97) % _N]).reshape(_NW, _NCH, _CH)
    dst_p = jnp.concatenate([dst, _N + ar % (_N_PAD - _N)]).reshape(
        _NW, _NCH, _CH)
    zeros64 = jnp.zeros((_RPT, _DH), jnp.float32)
    zeros16 = jnp.zeros((_RPT, 16), jnp.float32)
    ones16 = jnp.ones((_CH, 16), jnp.float32)
    batch_r = jnp.pad(batch.astype(jnp.int32), (0, _N_PAD - _N),
                      constant_values=_G).reshape(_NBLK, 1, _MB)
    blr0, blr1, blr2 = (b.reshape(1, _DH) for b in (bl0, bl1, bl2))
    bmr = bm.reshape(1, _DH)
    b1r = b1.reshape(1, _DH)
    b2r = b2.reshape(1, _DT)

    (wlm0, wrm0, wlm1, wrm1, wlm2, wrm2, bb0, bb1, bb2) = _wfold(
        Wl0, Wr0, Wl1, Wr1, Wl2, Wr2, Wm, blr0, blr1, blr2, bmr)

    cp = _cnt(dst_p, ones16, zeros16)

    a0, r0 = _mm2(x_p, wlm0, wrm0)
    p0 = _agg(a0, src_p, dst_p, zeros64)
    h1, a1, r1 = _cmb2(p0, cp, r0, bb0, wlm1, wrm1)

    p1 = _agg(a1, src_p, dst_p, zeros64)
    h2, a2, r2 = _cmb2(p1, cp, r1, bb1, wlm2, wrm2)

    p2 = _agg(a2, src_p, dst_p, zeros64)

    return _pool(batch_r, h1, h2, p2, cp, r2, bb2, W1, b1r, W2, b2r)


# speedup vs baseline: 16.2034x; 1.0125x over previous
import functools

import jax
import jax.numpy as jnp
from jax import lax
from jax.experimental import pallas as pl
from jax.experimental.pallas import tpu as pltpu
from jax.experimental.pallas import tpu_sc as plsc

_N = 10000
_E = 320000
_DF = 128
_DH = 64
_G = 64
_DT = 10

_NC = 2
_NS = 16
_NW = _NC * _NS

_N_PAD = 10112
_RPT = _N_PAD // _NS
_CH = 128
_EPT = ((_E // _NW + 2 * _CH - 1) // (2 * _CH)) * (2 * _CH)
_NCH = _EPT // _CH
_E_PAD = _EPT * _NW

_NBLK = 8
_MB = _N_PAD // _NBLK

_mesh = plsc.VectorSubcoreMesh(core_axis_name="c", subcore_axis_name="s")


_NBUF = 8
_NIF = 4
assert _NCH % _NBUF == 0 and _NIF <= _NBUF


def _make_agg(with_counts):
    def body(*args):
        if with_counts:
            (table, src, dst, zeros, ones, zeros16, out, cout, acc, src_v,
             dst_v, acc16, ones_v, *bufs) = args
        else:
            (table, src, dst, zeros, out, acc, src_v, dst_v, *bufs) = args
        rows = list(bufs[:_NBUF])
        gsem = list(bufs[_NBUF:2 * _NBUF])
        ssem = list(bufs[2 * _NBUF:3 * _NBUF])
        csem = bufs[3 * _NBUF] if with_counts else None
        c = lax.axis_index("c")
        s = lax.axis_index("s")
        wid = s * _NC + c
        pltpu.sync_copy(zeros, acc.at[pl.ds(s * _RPT, _RPT)])
        if with_counts:
            pltpu.sync_copy(zeros16, acc16.at[pl.ds(s * _RPT, _RPT)])
            pltpu.sync_copy(ones, ones_v)
        pltpu.sync_copy(src.at[wid], src_v)
        pltpu.sync_copy(dst.at[wid], dst_v)
        plsc.subcore_barrier()

        for k in range(_NIF):
            pltpu.async_copy(table.at[src_v.at[k]], rows[k], gsem[k])

        def step(i, carry):
            base = i * _NBUF
            for k in range(_NBUF):
                j = base + k
                jn = j + _NIF
                bn = (k + _NIF) % _NBUF

                @pl.when(jnp.logical_and(jn < _NCH, j >= _NBUF - _NIF))
                def _():
                    pltpu.make_async_copy(
                        rows[bn], acc.at[dst_v.at[jn - _NBUF]],
                        ssem[bn]).wait()

                @pl.when(jn < _NCH)
                def _():
                    pltpu.async_copy(table.at[src_v.at[jn]], rows[bn],
                                     gsem[bn])

                pltpu.make_async_copy(table.at[src_v.at[j]], rows[k],
                                      gsem[k]).wait()
                pltpu.async_copy(rows[k], acc.at[dst_v.at[j]], ssem[k],
                                 add=True)
                if with_counts:
                    @pl.when(j >= _NBUF)
                    def _():
                        pltpu.make_async_copy(
                            ones_v, acc16.at[dst_v.at[j - _NBUF]],
                            csem).wait()

                    pltpu.async_copy(ones_v, acc16.at[dst_v.at[j]], csem,
                                     add=True)
            return carry

        lax.fori_loop(0, _NCH // _NBUF, step, 0)
        for k in range(_NBUF):
            jt = _NCH - _NBUF + k
            pltpu.make_async_copy(rows[k], acc.at[dst_v.at[jt]],
                                  ssem[k]).wait()
            if with_counts:
                pltpu.make_async_copy(ones_v, acc16.at[dst_v.at[jt]],
                                      csem).wait()
        plsc.subcore_barrier()
        pltpu.sync_copy(acc.at[pl.ds(s * _RPT, _RPT)],
                        out.at[c, pl.ds(s * _RPT, _RPT)])
        if with_counts:
            pltpu.sync_copy(acc16.at[pl.ds(s * _RPT, _RPT)],
                            cout.at[c, pl.ds(s * _RPT, _RPT)])

    out_type = jax.ShapeDtypeStruct((_NC, _N_PAD, _DH), jnp.float32)
    scratch = [pltpu.VMEM_SHARED((_N_PAD, _DH), jnp.float32)]
    if with_counts:
        out_type = (out_type,
                    jax.ShapeDtypeStruct((_NC, _N_PAD, 16), jnp.float32))
    scratch += [
        pltpu.VMEM((_NCH, _CH), jnp.int32),
        pltpu.VMEM((_NCH, _CH), jnp.int32),
    ]
    if with_counts:
        scratch += [
            pltpu.VMEM_SHARED((_N_PAD, 16), jnp.float32),
            pltpu.VMEM((_CH, 16), jnp.float32),
        ]
    scratch += [pltpu.VMEM((_CH, _DH), jnp.float32)] * _NBUF
    scratch += [pltpu.SemaphoreType.DMA] * (2 * _NBUF + with_counts)
    return pl.kernel(
        body,
        out_type=out_type,
        mesh=_mesh,
        compiler_params=pltpu.CompilerParams(use_tc_tiling_on_sc=False),
        scratch_types=scratch,
    )


_agg = _make_agg(False)


def _cnt_body(dst, ones, zeros, out, acc, dst_v, ones_v, csem):
    c = lax.axis_index("c")
    s = lax.axis_index("s")
    wid = s * _NC + c
    pltpu.sync_copy(zeros, acc.at[pl.ds(s * _RPT, _RPT)])
    pltpu.sync_copy(dst.at[wid], dst_v)
    pltpu.sync_copy(ones, ones_v)
    plsc.subcore_barrier()

    def step(i, carry):
        base = i * _NBUF
        for k in range(_NBUF):
            j = base + k

            @pl.when(j >= _NBUF)
            def _():
                pltpu.make_async_copy(ones_v, acc.at[dst_v.at[j - _NBUF]],
                                      csem).wait()

            pltpu.async_copy(ones_v, acc.at[dst_v.at[j]], csem, add=True)
        return carry

    lax.fori_loop(0, _NCH // _NBUF, step, 0)
    for k in range(_NBUF):
        pltpu.make_async_copy(ones_v, acc.at[dst_v.at[_NCH - _NBUF + k]],
                              csem).wait()
    plsc.subcore_barrier()
    pltpu.sync_copy(acc.at[pl.ds(s * _RPT, _RPT)],
                    out.at[c, pl.ds(s * _RPT, _RPT)])


_cnt = pl.kernel(
    _cnt_body,
    out_type=jax.ShapeDtypeStruct((_NC, _N_PAD, 16), jnp.float32),
    mesh=_mesh,
    compiler_params=pltpu.CompilerParams(use_tc_tiling_on_sc=False),
    scratch_types=[
        pltpu.VMEM_SHARED((_N_PAD, 16), jnp.float32),
        pltpu.VMEM((_NCH, _CH), jnp.int32),
        pltpu.VMEM((_CH, 16), jnp.float32),
        pltpu.SemaphoreType.DMA,
    ],
)


def _wfold_kernel(wl0_ref, wr0_ref, wl1_ref, wr1_ref, wl2_ref, wr2_ref,
                  wm_ref, bl0_ref, bl1_ref, bl2_ref, bm_ref, *out_refs):
    wm = wm_ref[...]
    for k, w_ref in enumerate((wl0_ref, wr0_ref, wl1_ref, wr1_ref, wl2_ref,
                               wr2_ref)):
        out_refs[k][...] = jnp.dot(w_ref[...], wm,
                                   preferred_element_type=jnp.float32)
    bm = bm_ref[...]
    for k, b_ref in enumerate((bl0_ref, bl1_ref, bl2_ref)):
        out_refs[6 + k][...] = jnp.dot(b_ref[...], wm,
                                       preferred_element_type=jnp.float32) + bm


def _wfold(wl0, wr0, wl1, wr1, wl2, wr2, wm, bl0, bl1, bl2, bm):
    full = lambda shape: pl.BlockSpec(shape, lambda: (0, 0))
    return pl.pallas_call(
        _wfold_kernel,
        in_specs=[full((_DF, _DH)), full((_DF, _DH))] +
                 [full((_DH, _DH))] * 5 + [full((1, _DH))] * 4,
        out_specs=[full((_DF, _DH)), full((_DF, _DH))] +
                  [full((_DH, _DH))] * 4 + [full((1, _DH))] * 3,
        out_shape=[jax.ShapeDtypeStruct((_DF, _DH), jnp.float32)] * 2 +
                  [jax.ShapeDtypeStruct((_DH, _DH), jnp.float32)] * 4 +
                  [jax.ShapeDtypeStruct((1, _DH), jnp.float32)] * 3,
    )(wl0, wr0, wl1, wr1, wl2, wr2, wm, bl0, bl1, bl2, bm)


def _mm2_kernel(h_ref, wa_ref, wb_ref, a_ref, r_ref):
    h = h_ref[...]
    a_ref[...] = jnp.dot(h, wa_ref[...], preferred_element_type=jnp.float32)
    r_ref[...] = jnp.dot(h, wb_ref[...], preferred_element_type=jnp.float32)


def _mm2(h, wa, wb):
    d = h.shape[1]
    return pl.pallas_call(
        _mm2_kernel,
        grid=(_NBLK,),
        in_specs=[
            pl.BlockSpec((_MB, d), lambda i: (i, 0)),
            pl.BlockSpec((d, _DH), lambda i: (0, 0)),
            pl.BlockSpec((d, _DH), lambda i: (0, 0)),
        ],
        out_specs=[
            pl.BlockSpec((_MB, _DH), lambda i: (i, 0)),
            pl.BlockSpec((_MB, _DH), lambda i: (i, 0)),
        ],
        out_shape=[
            jax.ShapeDtypeStruct((_N_PAD, _DH), jnp.float32),
            jax.ShapeDtypeStruct((_N_PAD, _DH), jnp.float32),
        ],
    )(h, wa, wb)


def _hcomb(p, cp, r, bb):
    cnt = cp[0, :, 0] + cp[1, :, 0]
    rcp = 1.0 / jnp.clip(cnt, 1.0, None)
    return jnp.maximum((p[0] + p[1]) * rcp[:, None] + bb + r, 0.0)


def _cmb2_kernel(p_ref, cp_ref, r_ref, bb_ref, wa_ref, wb_ref, h_ref, a_ref,
                 r2_ref):
    h = _hcomb(p_ref[...], cp_ref[...], r_ref[...], bb_ref[...])
    h_ref[...] = h
    a_ref[...] = jnp.dot(h, wa_ref[...], preferred_element_type=jnp.float32)
    r2_ref[...] = jnp.dot(h, wb_ref[...], preferred_element_type=jnp.float32)


def _cmb2(p, cp, r, bb, wa, wb):
    return pl.pallas_call(
        _cmb2_kernel,
        grid=(_NBLK,),
        in_specs=[
            pl.BlockSpec((_NC, _MB, _DH), lambda i: (0, i, 0)),
            pl.BlockSpec((_NC, _MB, 16), lambda i: (0, i, 0)),
            pl.BlockSpec((_MB, _DH), lambda i: (i, 0)),
            pl.BlockSpec((1, _DH), lambda i: (0, 0)),
            pl.BlockSpec((_DH, _DH), lambda i: (0, 0)),
            pl.BlockSpec((_DH, _DH), lambda i: (0, 0)),
        ],
        out_specs=[
            pl.BlockSpec((_MB, _DH), lambda i: (i, 0)),
            pl.BlockSpec((_MB, _DH), lambda i: (i, 0)),
            pl.BlockSpec((_MB, _DH), lambda i: (i, 0)),
        ],
        out_shape=[jax.ShapeDtypeStruct((_N_PAD, _DH), jnp.float32)] * 3,
    )(p, cp, r, bb, wa, wb)


def _pool_kernel(b_ref, h1_ref, h2_ref, p_ref, cp_ref, r_ref, bb_ref,
                 w1_ref, b1_ref, w2_ref, b2_ref, o_ref, acc_ref):
    i = pl.program_id(0)

    @pl.when(i == 0)
    def _():
        acc_ref[...] = jnp.zeros_like(acc_ref)

    h3 = _hcomb(p_ref[...], cp_ref[...], r_ref[...], bb_ref[...])
    bb = b_ref[0, 0, :]
    onehot = (bb[:, None] ==
              lax.broadcasted_iota(jnp.int32, (1, _G), 1)).astype(jnp.float32)
    dn = (((0,), (0,)), ((), ()))
    for k, h in enumerate((h1_ref[...], h2_ref[...], h3)):
        acc_ref[:, k * _DH:(k + 1) * _DH] += lax.dot_general(
            onehot, h, dn, preferred_element_type=jnp.float32)
    acc_ref[:, 3 * _DH:3 * _DH + 1] += jnp.sum(onehot, axis=0)[:, None]

    @pl.when(i == _NBLK - 1)
    def _():
        accv = acc_ref[...]
        rcp = 1.0 / jnp.clip(accv[:, 3 * _DH:3 * _DH + 1], 1.0, None)
        pooled = accv[:, :3 * _DH] * rcp
        z = jnp.dot(pooled, w1_ref[...], preferred_element_type=jnp.float32)
        z = jnp.maximum(z + b1_ref[...], 0.0)
        o_ref[...] = jnp.dot(
            z, w2_ref[...], preferred_element_type=jnp.float32) + b2_ref[...]


def _pool(batch_r, h1, h2, p, cp, r, bb, w1, b1, w2, b2):
    return pl.pallas_call(
        _pool_kernel,
        grid=(_NBLK,),
        in_specs=[
            pl.BlockSpec((1, 1, _MB), lambda i: (i, 0, 0)),
            pl.BlockSpec((_MB, _DH), lambda i: (i, 0)),
            pl.BlockSpec((_MB, _DH), lambda i: (i, 0)),
            pl.BlockSpec((_NC, _MB, _DH), lambda i: (0, i, 0)),
            pl.BlockSpec((_NC, _MB, 16), lambda i: (0, i, 0)),
            pl.BlockSpec((_MB, _DH), lambda i: (i, 0)),
            pl.BlockSpec((1, _DH), lambda i: (0, 0)),
            pl.BlockSpec((3 * _DH, _DH), lambda i: (0, 0)),
            pl.BlockSpec((1, _DH), lambda i: (0, 0)),
            pl.BlockSpec((_DH, _DT), lambda i: (0, 0)),
            pl.BlockSpec((1, _DT), lambda i: (0, 0)),
        ],
        out_specs=pl.BlockSpec((_G, _DT), lambda i: (0, 0)),
        out_shape=jax.ShapeDtypeStruct((_G, _DT), jnp.float32),
        scratch_shapes=[pltpu.VMEM((_G, 3 * _DH + 128), jnp.float32)],
    )(batch_r, h1, h2, p, cp, r, bb, w1, b1, w2, b2)


def kernel(x, edge_index, batch, Wl0, bl0, Wr0, Wl1, bl1, Wr1, Wl2, bl2,
           Wr2, Wm, bm, W1, b1, W2, b2):
    x_p = jnp.pad(x, ((0, _N_PAD - _N), (0, 0)))
    src = edge_index[0].astype(jnp.int32)
    dst = edge_index[1].astype(jnp.int32)
    npad = _E_PAD - _E
    ar = jnp.arange(npad, dtype=jnp.int32)
    src_p = jnp.concatenate([src, (ar * 97) % _N]).reshape(_NW, _NCH, _CH)
    dst_p = jnp.concatenate([dst, _N + ar % (_N_PAD - _N)]).reshape(
        _NW, _NCH, _CH)
    zeros64 = jnp.zeros((_RPT, _DH), jnp.float32)
    zeros16 = jnp.zeros((_RPT, 16), jnp.float32)
    ones16 = jnp.ones((_CH, 16), jnp.float32)
    batch_r = jnp.pad(batch.astype(jnp.int32), (0, _N_PAD - _N),
                      constant_values=_G).reshape(_NBLK, 1, _MB)
    blr0, blr1, blr2 = (b.reshape(1, _DH) for b in (bl0, bl1, bl2))
    bmr = bm.reshape(1, _DH)
    b1r = b1.reshape(1, _DH)
    b2r = b2.reshape(1, _DT)

    (wlm0, wrm0, wlm1, wrm1, wlm2, wrm2, bb0, bb1, bb2) = _wfold(
        Wl0, Wr0, Wl1, Wr1, Wl2, Wr2, Wm, blr0, blr1, blr2, bmr)

    cp = _cnt(dst_p, ones16, zeros16)
    a0, r0 = _mm2(x_p, wlm0, wrm0)
    p0 = _agg(a0, src_p, dst_p, zeros64)
    h1, a1, r1 = _cmb2(p0, cp, r0, bb0, wlm1, wrm1)

    p1 = _agg(a1, src_p, dst_p, zeros64)
    h2, a2, r2 = _cmb2(p1, cp, r1, bb1, wlm2, wrm2)

    p2 = _agg(a2, src_p, dst_p, zeros64)

    return _pool(batch_r, h1, h2, p2, cp, r2, bb2, W1, b1r, W2, b2r)


# speedup vs baseline: 16.4264x; 1.0138x over previous
import functools

import jax
import jax.numpy as jnp
from jax import lax
from jax.experimental import pallas as pl
from jax.experimental.pallas import tpu as pltpu
from jax.experimental.pallas import tpu_sc as plsc

_N = 10000
_E = 320000
_DF = 128
_DH = 64
_G = 64
_DT = 10

_NC = 2
_NS = 16
_NW = _NC * _NS

_N_PAD = 10112
_RPT = _N_PAD // _NS
_CH = 256
_ECH = _CH
_EPT = ((_E // _NW + _ECH - 1) // _ECH) * _ECH
_NCH = _EPT // _ECH
_E_PAD = _EPT * _NW

_NBLK = 8
_MB = _N_PAD // _NBLK

_mesh = plsc.VectorSubcoreMesh(core_axis_name="c", subcore_axis_name="s")


_NBUF = 4
_NIF = 3
assert _NCH % _NBUF == 0 and _NIF <= _NBUF


def _make_agg(with_counts):
    def body(*args):
        if with_counts:
            (table, src, dst, zeros, ones, zeros16, out, cout, acc, src_v,
             dst_v, acc16, ones_v, *bufs) = args
        else:
            (table, src, dst, zeros, out, acc, src_v, dst_v, *bufs) = args
        rows = list(bufs[:_NBUF])
        gsem = list(bufs[_NBUF:2 * _NBUF])
        ssem = list(bufs[2 * _NBUF:3 * _NBUF])
        csem = bufs[3 * _NBUF] if with_counts else None
        c = lax.axis_index("c")
        s = lax.axis_index("s")
        wid = s * _NC + c
        pltpu.sync_copy(zeros, acc.at[pl.ds(s * _RPT, _RPT)])
        if with_counts:
            pltpu.sync_copy(zeros16, acc16.at[pl.ds(s * _RPT, _RPT)])
            pltpu.sync_copy(ones, ones_v)
        pltpu.sync_copy(src.at[wid], src_v)
        pltpu.sync_copy(dst.at[wid], dst_v)
        plsc.subcore_barrier()

        for k in range(_NIF):
            pltpu.async_copy(table.at[src_v.at[k]], rows[k], gsem[k])

        def step(i, carry):
            base = i * _NBUF
            for k in range(_NBUF):
                j = base + k
                jn = j + _NIF
                bn = (k + _NIF) % _NBUF

                @pl.when(jnp.logical_and(jn < _NCH, j >= _NBUF - _NIF))
                def _():
                    pltpu.make_async_copy(
                        rows[bn], acc.at[dst_v.at[jn - _NBUF]],
                        ssem[bn]).wait()

                @pl.when(jn < _NCH)
                def _():
                    pltpu.async_copy(table.at[src_v.at[jn]], rows[bn],
                                     gsem[bn])

                pltpu.make_async_copy(table.at[src_v.at[j]], rows[k],
                                      gsem[k]).wait()
                pltpu.async_copy(rows[k], acc.at[dst_v.at[j]], ssem[k],
                                 add=True)
                if with_counts:
                    @pl.when(j >= _NBUF)
                    def _():
                        pltpu.make_async_copy(
                            ones_v, acc16.at[dst_v.at[j - _NBUF]],
                            csem).wait()

                    pltpu.async_copy(ones_v, acc16.at[dst_v.at[j]], csem,
                                     add=True)
            return carry

        lax.fori_loop(0, _NCH // _NBUF, step, 0)
        for k in range(_NBUF):
            jt = _NCH - _NBUF + k
            pltpu.make_async_copy(rows[k], acc.at[dst_v.at[jt]],
                                  ssem[k]).wait()
            if with_counts:
                pltpu.make_async_copy(ones_v, acc16.at[dst_v.at[jt]],
                                      csem).wait()
        plsc.subcore_barrier()
        pltpu.sync_copy(acc.at[pl.ds(s * _RPT, _RPT)],
                        out.at[c, pl.ds(s * _RPT, _RPT)])
        if with_counts:
            pltpu.sync_copy(acc16.at[pl.ds(s * _RPT, _RPT)],
                            cout.at[c, pl.ds(s * _RPT, _RPT)])

    out_type = jax.ShapeDtypeStruct((_NC, _N_PAD, _DH), jnp.float32)
    scratch = [pltpu.VMEM_SHARED((_N_PAD, _DH), jnp.float32)]
    if with_counts:
        out_type = (out_type,
                    jax.ShapeDtypeStruct((_NC, _N_PAD, 16), jnp.float32))
    scratch += [
        pltpu.VMEM((_NCH, _CH), jnp.int32),
        pltpu.VMEM((_NCH, _CH), jnp.int32),
    ]
    if with_counts:
        scratch += [
            pltpu.VMEM_SHARED((_N_PAD, 16), jnp.float32),
            pltpu.VMEM((_CH, 16), jnp.float32),
        ]
    scratch += [pltpu.VMEM((_CH, _DH), jnp.float32)] * _NBUF
    scratch += [pltpu.SemaphoreType.DMA] * (2 * _NBUF + with_counts)
    return pl.kernel(
        body,
        out_type=out_type,
        mesh=_mesh,
        compiler_params=pltpu.CompilerParams(use_tc_tiling_on_sc=False),
        scratch_types=scratch,
    )


_agg = _make_agg(False)


def _cnt_body(dst, ones, zeros, out, acc, dst_v, ones_v, csem):
    c = lax.axis_index("c")
    s = lax.axis_index("s")
    wid = s * _NC + c
    pltpu.sync_copy(zeros, acc.at[pl.ds(s * _RPT, _RPT)])
    pltpu.sync_copy(dst.at[wid], dst_v)
    pltpu.sync_copy(ones, ones_v)
    plsc.subcore_barrier()

    def step(i, carry):
        base = i * _NBUF
        for k in range(_NBUF):
            j = base + k

            @pl.when(j >= _NBUF)
            def _():
                pltpu.make_async_copy(ones_v, acc.at[dst_v.at[j - _NBUF]],
                                      csem).wait()

            pltpu.async_copy(ones_v, acc.at[dst_v.at[j]], csem, add=True)
        return carry

    lax.fori_loop(0, _NCH // _NBUF, step, 0)
    for k in range(_NBUF):
        pltpu.make_async_copy(ones_v, acc.at[dst_v.at[_NCH - _NBUF + k]],
                              csem).wait()
    plsc.subcore_barrier()
    pltpu.sync_copy(acc.at[pl.ds(s * _RPT, _RPT)],
                    out.at[c, pl.ds(s * _RPT, _RPT)])


_cnt = pl.kernel(
    _cnt_body,
    out_type=jax.ShapeDtypeStruct((_NC, _N_PAD, 16), jnp.float32),
    mesh=_mesh,
    compiler_params=pltpu.CompilerParams(use_tc_tiling_on_sc=False),
    scratch_types=[
        pltpu.VMEM_SHARED((_N_PAD, 16), jnp.float32),
        pltpu.VMEM((_NCH, _CH), jnp.int32),
        pltpu.VMEM((_CH, 16), jnp.float32),
        pltpu.SemaphoreType.DMA,
    ],
)


def _wfold_kernel(wl0_ref, wr0_ref, wl1_ref, wr1_ref, wl2_ref, wr2_ref,
                  wm_ref, bl0_ref, bl1_ref, bl2_ref, bm_ref, *out_refs):
    wm = wm_ref[...]
    for k, w_ref in enumerate((wl0_ref, wr0_ref, wl1_ref, wr1_ref, wl2_ref,
                               wr2_ref)):
        out_refs[k][...] = jnp.dot(w_ref[...], wm,
                                   preferred_element_type=jnp.float32)
    bm = bm_ref[...]
    for k, b_ref in enumerate((bl0_ref, bl1_ref, bl2_ref)):
        out_refs[6 + k][...] = jnp.dot(b_ref[...], wm,
                                       preferred_element_type=jnp.float32) + bm


def _wfold(wl0, wr0, wl1, wr1, wl2, wr2, wm, bl0, bl1, bl2, bm):
    full = lambda shape: pl.BlockSpec(shape, lambda: (0, 0))
    return pl.pallas_call(
        _wfold_kernel,
        in_specs=[full((_DF, _DH)), full((_DF, _DH))] +
                 [full((_DH, _DH))] * 5 + [full((1, _DH))] * 4,
        out_specs=[full((_DF, _DH)), full((_DF, _DH))] +
                  [full((_DH, _DH))] * 4 + [full((1, _DH))] * 3,
        out_shape=[jax.ShapeDtypeStruct((_DF, _DH), jnp.float32)] * 2 +
                  [jax.ShapeDtypeStruct((_DH, _DH), jnp.float32)] * 4 +
                  [jax.ShapeDtypeStruct((1, _DH), jnp.float32)] * 3,
    )(wl0, wr0, wl1, wr1, wl2, wr2, wm, bl0, bl1, bl2, bm)


def _mm2_kernel(h_ref, wa_ref, wb_ref, a_ref, r_ref):
    h = h_ref[...]
    a_ref[...] = jnp.dot(h, wa_ref[...], preferred_element_type=jnp.float32)
    r_ref[...] = jnp.dot(h, wb_ref[...], preferred_element_type=jnp.float32)


def _mm2(h, wa, wb):
    d = h.shape[1]
    return pl.pallas_call(
        _mm2_kernel,
        grid=(_NBLK,),
        in_specs=[
            pl.BlockSpec((_MB, d), lambda i: (i, 0)),
            pl.BlockSpec((d, _DH), lambda i: (0, 0)),
            pl.BlockSpec((d, _DH), lambda i: (0, 0)),
        ],
        out_specs=[
            pl.BlockSpec((_MB, _DH), lambda i: (i, 0)),
            pl.BlockSpec((_MB, _DH), lambda i: (i, 0)),
        ],
        out_shape=[
            jax.ShapeDtypeStruct((_N_PAD, _DH), jnp.float32),
            jax.ShapeDtypeStruct((_N_PAD, _DH), jnp.float32),
        ],
    )(h, wa, wb)


def _hcomb(p, cp, r, bb):
    cnt = cp[0, :, 0] + cp[1, :, 0]
    rcp = 1.0 / jnp.clip(cnt, 1.0, None)
    return jnp.maximum((p[0] + p[1]) * rcp[:, None] + bb + r, 0.0)


def _cmb2_kernel(p_ref, cp_ref, r_ref, bb_ref, wa_ref, wb_ref, h_ref, a_ref,
                 r2_ref):
    h = _hcomb(p_ref[...], cp_ref[...], r_ref[...], bb_ref[...])
    h_ref[...] = h
    a_ref[...] = jnp.dot(h, wa_ref[...], preferred_element_type=jnp.float32)
    r2_ref[...] = jnp.dot(h, wb_ref[...], preferred_element_type=jnp.float32)


def _cmb2(p, cp, r, bb, wa, wb):
    return pl.pallas_call(
        _cmb2_kernel,
        grid=(_NBLK,),
        in_specs=[
            pl.BlockSpec((_NC, _MB, _DH), lambda i: (0, i, 0)),
            pl.BlockSpec((_NC, _MB, 16), lambda i: (0, i, 0)),
            pl.BlockSpec((_MB, _DH), lambda i: (i, 0)),
            pl.BlockSpec((1, _DH), lambda i: (0, 0)),
            pl.BlockSpec((_DH, _DH), lambda i: (0, 0)),
            pl.BlockSpec((_DH, _DH), lambda i: (0, 0)),
        ],
        out_specs=[
            pl.BlockSpec((_MB, _DH), lambda i: (i, 0)),
            pl.BlockSpec((_MB, _DH), lambda i: (i, 0)),
            pl.BlockSpec((_MB, _DH), lambda i: (i, 0)),
        ],
        out_shape=[jax.ShapeDtypeStruct((_N_PAD, _DH), jnp.float32)] * 3,
    )(p, cp, r, bb, wa, wb)


def _pool_kernel(b_ref, h1_ref, h2_ref, p_ref, cp_ref, r_ref, bb_ref,
                 w1_ref, b1_ref, w2_ref, b2_ref, o_ref, acc_ref):
    i = pl.program_id(0)

    @pl.when(i == 0)
    def _():
        acc_ref[...] = jnp.zeros_like(acc_ref)

    h3 = _hcomb(p_ref[...], cp_ref[...], r_ref[...], bb_ref[...])
    bb = b_ref[0, 0, :]
    onehot = (bb[:, None] ==
              lax.broadcasted_iota(jnp.int32, (1, _G), 1)).astype(jnp.float32)
    dn = (((0,), (0,)), ((), ()))
    for k, h in enumerate((h1_ref[...], h2_ref[...], h3)):
        acc_ref[:, k * _DH:(k + 1) * _DH] += lax.dot_general(
            onehot, h, dn, preferred_element_type=jnp.float32)
    acc_ref[:, 3 * _DH:3 * _DH + 1] += jnp.sum(onehot, axis=0)[:, None]

    @pl.when(i == _NBLK - 1)
    def _():
        accv = acc_ref[...]
        rcp = 1.0 / jnp.clip(accv[:, 3 * _DH:3 * _DH + 1], 1.0, None)
        pooled = accv[:, :3 * _DH] * rcp
        z = jnp.dot(pooled, w1_ref[...], preferred_element_type=jnp.float32)
        z = jnp.maximum(z + b1_ref[...], 0.0)
        o_ref[...] = jnp.dot(
            z, w2_ref[...], preferred_element_type=jnp.float32) + b2_ref[...]


def _pool(batch_r, h1, h2, p, cp, r, bb, w1, b1, w2, b2):
    return pl.pallas_call(
        _pool_kernel,
        grid=(_NBLK,),
        in_specs=[
            pl.BlockSpec((1, 1, _MB), lambda i: (i, 0, 0)),
            pl.BlockSpec((_MB, _DH), lambda i: (i, 0)),
            pl.BlockSpec((_MB, _DH), lambda i: (i, 0)),
            pl.BlockSpec((_NC, _MB, _DH), lambda i: (0, i, 0)),
            pl.BlockSpec((_NC, _MB, 16), lambda i: (0, i, 0)),
            pl.BlockSpec((_MB, _DH), lambda i: (i, 0)),
            pl.BlockSpec((1, _DH), lambda i: (0, 0)),
            pl.BlockSpec((3 * _DH, _DH), lambda i: (0, 0)),
            pl.BlockSpec((1, _DH), lambda i: (0, 0)),
            pl.BlockSpec((_DH, _DT), lambda i: (0, 0)),
            pl.BlockSpec((1, _DT), lambda i: (0, 0)),
        ],
        out_specs=pl.BlockSpec((_G, _DT), lambda i: (0, 0)),
        out_shape=jax.ShapeDtypeStruct((_G, _DT), jnp.float32),
        scratch_shapes=[pltpu.VMEM((_G, 3 * _DH + 128), jnp.float32)],
    )(batch_r, h1, h2, p, cp, r, bb, w1, b1, w2, b2)


def kernel(x, edge_index, batch, Wl0, bl0, Wr0, Wl1, bl1, Wr1, Wl2, bl2,
           Wr2, Wm, bm, W1, b1, W2, b2):
    x_p = jnp.pad(x, ((0, _N_PAD - _N), (0, 0)))
    src = edge_index[0].astype(jnp.int32)
    dst = edge_index[1].astype(jnp.int32)
    npad = _E_PAD - _E
    ar = jnp.arange(npad, dtype=jnp.int32)
    src_p = jnp.concatenate([src, (ar * 97) % _N]).reshape(_NW, _NCH, _CH)
    dst_p = jnp.concatenate([dst, _N + ar % (_N_PAD - _N)]).reshape(_NW, _NCH, _CH)
    zeros64 = jnp.zeros((_RPT, _DH), jnp.float32)
    zeros16 = jnp.zeros((_RPT, 16), jnp.float32)
    ones16 = jnp.ones((_CH, 16), jnp.float32)
    batch_r = jnp.pad(batch.astype(jnp.int32), (0, _N_PAD - _N),
                      constant_values=_G).reshape(_NBLK, 1, _MB)
    blr0, blr1, blr2 = (b.reshape(1, _DH) for b in (bl0, bl1, bl2))
    bmr = bm.reshape(1, _DH)
    b1r = b1.reshape(1, _DH)
    b2r = b2.reshape(1, _DT)

    (wlm0, wrm0, wlm1, wrm1, wlm2, wrm2, bb0, bb1, bb2) = _wfold(
        Wl0, Wr0, Wl1, Wr1, Wl2, Wr2, Wm, blr0, blr1, blr2, bmr)

    cp = _cnt(dst_p, ones16, zeros16)
    a0, r0 = _mm2(x_p, wlm0, wrm0)
    p0 = _agg(a0, src_p, dst_p, zeros64)
    h1, a1, r1 = _cmb2(p0, cp, r0, bb0, wlm1, wrm1)

    p1 = _agg(a1, src_p, dst_p, zeros64)
    h2, a2, r2 = _cmb2(p1, cp, r1, bb1, wlm2, wrm2)

    p2 = _agg(a2, src_p, dst_p, zeros64)

    return _pool(batch_r, h1, h2, p2, cp, r2, bb2, W1, b1r, W2, b2r)


# speedup vs baseline: 16.7372x; 1.0189x over previous
import functools

import jax
import jax.numpy as jnp
from jax import lax
from jax.experimental import pallas as pl
from jax.experimental.pallas import tpu as pltpu
from jax.experimental.pallas import tpu_sc as plsc

_N = 10000
_E = 320000
_DF = 128
_DH = 64
_G = 64
_DT = 10

_NC = 2
_NS = 16
_NW = _NC * _NS

_N_PAD = 10112
_RPT = _N_PAD // _NS
_CH = 256
_ECH = _CH
_EPT = ((_E // _NW + _ECH - 1) // _ECH) * _ECH
_NCH = _EPT // _ECH
_E_PAD = _EPT * _NW

_NBLK = 4
_MB = _N_PAD // _NBLK

_mesh = plsc.VectorSubcoreMesh(core_axis_name="c", subcore_axis_name="s")


_NBUF = 4
_NIF = 3
assert _NCH % _NBUF == 0 and _NIF <= _NBUF


def _make_agg(with_counts):
    def body(*args):
        if with_counts:
            (table, src, dst, zeros, ones, zeros16, out, cout, acc, src_v,
             dst_v, acc16, ones_v, *bufs) = args
        else:
            (table, src, dst, zeros, out, acc, src_v, dst_v, *bufs) = args
        rows = list(bufs[:_NBUF])
        gsem = list(bufs[_NBUF:2 * _NBUF])
        ssem = list(bufs[2 * _NBUF:3 * _NBUF])
        csem = bufs[3 * _NBUF] if with_counts else None
        c = lax.axis_index("c")
        s = lax.axis_index("s")
        wid = s * _NC + c
        pltpu.sync_copy(zeros, acc.at[pl.ds(s * _RPT, _RPT)])
        if with_counts:
            pltpu.sync_copy(zeros16, acc16.at[pl.ds(s * _RPT, _RPT)])
            pltpu.sync_copy(ones, ones_v)
        pltpu.sync_copy(src.at[wid], src_v)
        pltpu.sync_copy(dst.at[wid], dst_v)
        plsc.subcore_barrier()

        for k in range(_NIF):
            pltpu.async_copy(table.at[src_v.at[k]], rows[k], gsem[k])

        def step(i, carry):
            base = i * _NBUF
            for k in range(_NBUF):
                j = base + k
                jn = j + _NIF
                bn = (k + _NIF) % _NBUF

                @pl.when(jnp.logical_and(jn < _NCH, j >= _NBUF - _NIF))
                def _():
                    pltpu.make_async_copy(
                        rows[bn], acc.at[dst_v.at[jn - _NBUF]],
                        ssem[bn]).wait()

                @pl.when(jn < _NCH)
                def _():
                    pltpu.async_copy(table.at[src_v.at[jn]], rows[bn],
                                     gsem[bn])

                pltpu.make_async_copy(table.at[src_v.at[j]], rows[k],
                                      gsem[k]).wait()
                pltpu.async_copy(rows[k], acc.at[dst_v.at[j]], ssem[k],
                                 add=True)
                if with_counts:
                    @pl.when(j >= _NBUF)
                    def _():
                        pltpu.make_async_copy(
                            ones_v, acc16.at[dst_v.at[j - _NBUF]],
                            csem).wait()

                    pltpu.async_copy(ones_v, acc16.at[dst_v.at[j]], csem,
                                     add=True)
            return carry

        lax.fori_loop(0, _NCH // _NBUF, step, 0)
        for k in range(_NBUF):
            jt = _NCH - _NBUF + k
            pltpu.make_async_copy(rows[k], acc.at[dst_v.at[jt]],
                                  ssem[k]).wait()
            if with_counts:
                pltpu.make_async_copy(ones_v, acc16.at[dst_v.at[jt]],
                                      csem).wait()
        plsc.subcore_barrier()
        pltpu.sync_copy(acc.at[pl.ds(s * _RPT, _RPT)],
                        out.at[c, pl.ds(s * _RPT, _RPT)])
        if with_counts:
            pltpu.sync_copy(acc16.at[pl.ds(s * _RPT, _RPT)],
                            cout.at[c, pl.ds(s * _RPT, _RPT)])

    out_type = jax.ShapeDtypeStruct((_NC, _N_PAD, _DH), jnp.float32)
    scratch = [pltpu.VMEM_SHARED((_N_PAD, _DH), jnp.float32)]
    if with_counts:
        out_type = (out_type,
                    jax.ShapeDtypeStruct((_NC, _N_PAD, 16), jnp.float32))
    scratch += [
        pltpu.VMEM((_NCH, _CH), jnp.int32),
        pltpu.VMEM((_NCH, _CH), jnp.int32),
    ]
    if with_counts:
        scratch += [
            pltpu.VMEM_SHARED((_N_PAD, 16), jnp.float32),
            pltpu.VMEM((_CH, 16), jnp.float32),
        ]
    scratch += [pltpu.VMEM((_CH, _DH), jnp.float32)] * _NBUF
    scratch += [pltpu.SemaphoreType.DMA] * (2 * _NBUF + with_counts)
    return pl.kernel(
        body,
        out_type=out_type,
        mesh=_mesh,
        compiler_params=pltpu.CompilerParams(use_tc_tiling_on_sc=False),
        scratch_types=scratch,
    )


_agg = _make_agg(False)


def _cnt_body(dst, ones, zeros, out, acc, dst_v, ones_v, csem):
    c = lax.axis_index("c")
    s = lax.axis_index("s")
    wid = s * _NC + c
    pltpu.sync_copy(zeros, acc.at[pl.ds(s * _RPT, _RPT)])
    pltpu.sync_copy(dst.at[wid], dst_v)
    pltpu.sync_copy(ones, ones_v)
    plsc.subcore_barrier()

    def step(i, carry):
        base = i * _NBUF
        for k in range(_NBUF):
            j = base + k

            @pl.when(j >= _NBUF)
            def _():
                pltpu.make_async_copy(ones_v, acc.at[dst_v.at[j - _NBUF]],
                                      csem).wait()

            pltpu.async_copy(ones_v, acc.at[dst_v.at[j]], csem, add=True)
        return carry

    lax.fori_loop(0, _NCH // _NBUF, step, 0)
    for k in range(_NBUF):
        pltpu.make_async_copy(ones_v, acc.at[dst_v.at[_NCH - _NBUF + k]],
                              csem).wait()
    plsc.subcore_barrier()
    pltpu.sync_copy(acc.at[pl.ds(s * _RPT, _RPT)],
                    out.at[c, pl.ds(s * _RPT, _RPT)])


_cnt = pl.kernel(
    _cnt_body,
    out_type=jax.ShapeDtypeStruct((_NC, _N_PAD, 16), jnp.float32),
    mesh=_mesh,
    compiler_params=pltpu.CompilerParams(use_tc_tiling_on_sc=False),
    scratch_types=[
        pltpu.VMEM_SHARED((_N_PAD, 16), jnp.float32),
        pltpu.VMEM((_NCH, _CH), jnp.int32),
        pltpu.VMEM((_CH, 16), jnp.float32),
        pltpu.SemaphoreType.DMA,
    ],
)


def _wfold_kernel(wl0_ref, wr0_ref, wl1_ref, wr1_ref, wl2_ref, wr2_ref,
                  wm_ref, bl0_ref, bl1_ref, bl2_ref, bm_ref, *out_refs):
    wm = wm_ref[...]
    for k, (wl_ref, wr_ref) in enumerate(
            ((wl0_ref, wr0_ref), (wl1_ref, wr1_ref), (wl2_ref, wr2_ref))):
        out_refs[k][:, :_DH] = jnp.dot(wl_ref[...], wm,
                                       preferred_element_type=jnp.float32)
        out_refs[k][:, _DH:] = jnp.dot(wr_ref[...], wm,
                                       preferred_element_type=jnp.float32)
    bm = bm_ref[...]
    for k, b_ref in enumerate((bl0_ref, bl1_ref, bl2_ref)):
        out_refs[3 + k][...] = jnp.dot(b_ref[...], wm,
                                       preferred_element_type=jnp.float32) + bm


def _wfold(wl0, wr0, wl1, wr1, wl2, wr2, wm, bl0, bl1, bl2, bm):
    full = lambda shape: pl.BlockSpec(shape, lambda: (0, 0))
    return pl.pallas_call(
        _wfold_kernel,
        in_specs=[full((_DF, _DH)), full((_DF, _DH))] +
                 [full((_DH, _DH))] * 5 + [full((1, _DH))] * 4,
        out_specs=[full((_DF, 2 * _DH))] + [full((_DH, 2 * _DH))] * 2 +
                  [full((1, _DH))] * 3,
        out_shape=[jax.ShapeDtypeStruct((_DF, 2 * _DH), jnp.float32)] +
                  [jax.ShapeDtypeStruct((_DH, 2 * _DH), jnp.float32)] * 2 +
                  [jax.ShapeDtypeStruct((1, _DH), jnp.float32)] * 3,
    )(wl0, wr0, wl1, wr1, wl2, wr2, wm, bl0, bl1, bl2, bm)


def _mm2_kernel(h_ref, wab_ref, a_ref, r_ref):
    z = jnp.dot(h_ref[...], wab_ref[...], preferred_element_type=jnp.float32)
    a_ref[...] = z[:, :_DH]
    r_ref[...] = z[:, _DH:]


def _mm2(h, wab):
    d = h.shape[1]
    return pl.pallas_call(
        _mm2_kernel,
        grid=(_NBLK,),
        in_specs=[
            pl.BlockSpec((_MB, d), lambda i: (i, 0)),
            pl.BlockSpec((d, 2 * _DH), lambda i: (0, 0)),
        ],
        out_specs=[
            pl.BlockSpec((_MB, _DH), lambda i: (i, 0)),
            pl.BlockSpec((_MB, _DH), lambda i: (i, 0)),
        ],
        out_shape=[
            jax.ShapeDtypeStruct((_N_PAD, _DH), jnp.float32),
            jax.ShapeDtypeStruct((_N_PAD, _DH), jnp.float32),
        ],
    )(h, wab)


def _hcomb(p, cp, r, bb):
    cnt = cp[0, :, 0] + cp[1, :, 0]
    rcp = 1.0 / jnp.clip(cnt, 1.0, None)
    return jnp.maximum((p[0] + p[1]) * rcp[:, None] + bb + r, 0.0)


def _cmb2_kernel(p_ref, cp_ref, r_ref, bb_ref, wab_ref, h_ref, a_ref,
                 r2_ref):
    h = _hcomb(p_ref[...], cp_ref[...], r_ref[...], bb_ref[...])
    h_ref[...] = h
    z = jnp.dot(h, wab_ref[...], preferred_element_type=jnp.float32)
    a_ref[...] = z[:, :_DH]
    r2_ref[...] = z[:, _DH:]


def _cmb2(p, cp, r, bb, wab):
    return pl.pallas_call(
        _cmb2_kernel,
        grid=(_NBLK,),
        in_specs=[
            pl.BlockSpec((_NC, _MB, _DH), lambda i: (0, i, 0)),
            pl.BlockSpec((_NC, _MB, 16), lambda i: (0, i, 0)),
            pl.BlockSpec((_MB, _DH), lambda i: (i, 0)),
            pl.BlockSpec((1, _DH), lambda i: (0, 0)),
            pl.BlockSpec((_DH, 2 * _DH), lambda i: (0, 0)),
        ],
        out_specs=[
            pl.BlockSpec((_MB, _DH), lambda i: (i, 0)),
            pl.BlockSpec((_MB, _DH), lambda i: (i, 0)),
            pl.BlockSpec((_MB, _DH), lambda i: (i, 0)),
        ],
        out_shape=[jax.ShapeDtypeStruct((_N_PAD, _DH), jnp.float32)] * 3,
    )(p, cp, r, bb, wab)


def _pool_kernel(b_ref, h1_ref, h2_ref, p_ref, cp_ref, r_ref, bb_ref,
                 w1_ref, b1_ref, w2_ref, b2_ref, o_ref, acc_ref):
    i = pl.program_id(0)

    @pl.when(i == 0)
    def _():
        acc_ref[...] = jnp.zeros_like(acc_ref)

    h3 = _hcomb(p_ref[...], cp_ref[...], r_ref[...], bb_ref[...])
    bb = b_ref[0, 0, :]
    onehot = (bb[:, None] ==
              lax.broadcasted_iota(jnp.int32, (1, _G), 1)).astype(jnp.float32)
    dn = (((0,), (0,)), ((), ()))
    for k, h in enumerate((h1_ref[...], h2_ref[...], h3)):
        acc_ref[:, k * _DH:(k + 1) * _DH] += lax.dot_general(
            onehot, h, dn, preferred_element_type=jnp.float32)
    acc_ref[:, 3 * _DH:3 * _DH + 1] += jnp.sum(onehot, axis=0)[:, None]

    @pl.when(i == _NBLK - 1)
    def _():
        accv = acc_ref[...]
        rcp = 1.0 / jnp.clip(accv[:, 3 * _DH:3 * _DH + 1], 1.0, None)
        pooled = accv[:, :3 * _DH] * rcp
        z = jnp.dot(pooled, w1_ref[...], preferred_element_type=jnp.float32)
        z = jnp.maximum(z + b1_ref[...], 0.0)
        o_ref[...] = jnp.dot(
            z, w2_ref[...], preferred_element_type=jnp.float32) + b2_ref[...]


def _pool(batch_r, h1, h2, p, cp, r, bb, w1, b1, w2, b2):
    return pl.pallas_call(
        _pool_kernel,
        grid=(_NBLK,),
        in_specs=[
            pl.BlockSpec((1, 1, _MB), lambda i: (i, 0, 0)),
            pl.BlockSpec((_MB, _DH), lambda i: (i, 0)),
            pl.BlockSpec((_MB, _DH), lambda i: (i, 0)),
            pl.BlockSpec((_NC, _MB, _DH), lambda i: (0, i, 0)),
            pl.BlockSpec((_NC, _MB, 16), lambda i: (0, i, 0)),
            pl.BlockSpec((_MB, _DH), lambda i: (i, 0)),
            pl.BlockSpec((1, _DH), lambda i: (0, 0)),
            pl.BlockSpec((3 * _DH, _DH), lambda i: (0, 0)),
            pl.BlockSpec((1, _DH), lambda i: (0, 0)),
            pl.BlockSpec((_DH, _DT), lambda i: (0, 0)),
            pl.BlockSpec((1, _DT), lambda i: (0, 0)),
        ],
        out_specs=pl.BlockSpec((_G, _DT), lambda i: (0, 0)),
        out_shape=jax.ShapeDtypeStruct((_G, _DT), jnp.float32),
        scratch_shapes=[pltpu.VMEM((_G, 3 * _DH + 128), jnp.float32)],
    )(batch_r, h1, h2, p, cp, r, bb, w1, b1, w2, b2)


def kernel(x, edge_index, batch, Wl0, bl0, Wr0, Wl1, bl1, Wr1, Wl2, bl2,
           Wr2, Wm, bm, W1, b1, W2, b2):
    x_p = jnp.pad(x, ((0, _N_PAD - _N), (0, 0)))
    src = edge_index[0].astype(jnp.int32)
    dst = edge_index[1].astype(jnp.int32)
    npad = _E_PAD - _E
    ar = jnp.arange(npad, dtype=jnp.int32)
    src_p = jnp.concatenate([src, (ar * 97) % _N]).reshape(_NW, _NCH, _CH)
    dst_p = jnp.concatenate([dst, _N + ar % (_N_PAD - _N)]).reshape(_NW, _NCH, _CH)
    zeros64 = jnp.zeros((_RPT, _DH), jnp.float32)
    zeros16 = jnp.zeros((_RPT, 16), jnp.float32)
    ones16 = jnp.ones((_CH, 16), jnp.float32)
    batch_r = jnp.pad(batch.astype(jnp.int32), (0, _N_PAD - _N),
                      constant_values=_G).reshape(_NBLK, 1, _MB)
    blr0, blr1, blr2 = (b.reshape(1, _DH) for b in (bl0, bl1, bl2))
    bmr = bm.reshape(1, _DH)
    b1r = b1.reshape(1, _DH)
    b2r = b2.reshape(1, _DT)

    (wab0, wab1, wab2, bb0, bb1, bb2) = _wfold(
        Wl0, Wr0, Wl1, Wr1, Wl2, Wr2, Wm, blr0, blr1, blr2, bmr)

    cp = _cnt(dst_p, ones16, zeros16)
    a0, r0 = _mm2(x_p, wab0)
    p0 = _agg(a0, src_p, dst_p, zeros64)
    h1, a1, r1 = _cmb2(p0, cp, r0, bb0, wab1)

    p1 = _agg(a1, src_p, dst_p, zeros64)
    h2, a2, r2 = _cmb2(p1, cp, r1, bb1, wab2)

    p2 = _agg(a2, src_p, dst_p, zeros64)

    return _pool(batch_r, h1, h2, p2, cp, r2, bb2, W1, b1r, W2, b2r)


# speedup vs baseline: 18.2193x; 1.0886x over previous
import functools

import jax
import jax.numpy as jnp
from jax import lax
from jax.experimental import pallas as pl
from jax.experimental.pallas import tpu as pltpu
from jax.experimental.pallas import tpu_sc as plsc

_N = 10000
_E = 320000
_DF = 128
_DH = 64
_G = 64
_DT = 10

_NC = 2
_NS = 16
_NW = _NC * _NS

_N_PAD = 10112
_RPT = _N_PAD // _NS
_CH = 256
_ECH = _CH
_EPT = ((_E // _NW + _ECH - 1) // _ECH) * _ECH
_NCH = _EPT // _ECH
_E_PAD = _EPT * _NW

_NBLK = 2
_MB = _N_PAD // _NBLK

_mesh = plsc.VectorSubcoreMesh(core_axis_name="c", subcore_axis_name="s")


_NBUF = 4
_NIF = 3
assert _NCH % _NBUF == 0 and _NIF <= _NBUF


def _make_agg(with_counts):
    def body(*args):
        if with_counts:
            (table, src, dst, zeros, ones, zeros16, out, cout, acc, src_v,
             dst_v, acc16, ones_v, *bufs) = args
        else:
            (table, src, dst, zeros, out, acc, src_v, dst_v, *bufs) = args
        rows = list(bufs[:_NBUF])
        gsem = list(bufs[_NBUF:2 * _NBUF])
        ssem = list(bufs[2 * _NBUF:3 * _NBUF])
        csem = bufs[3 * _NBUF] if with_counts else None
        c = lax.axis_index("c")
        s = lax.axis_index("s")
        wid = s * _NC + c
        pltpu.sync_copy(zeros, acc.at[pl.ds(s * _RPT, _RPT)])
        if with_counts:
            pltpu.sync_copy(zeros16, acc16.at[pl.ds(s * _RPT, _RPT)])
            pltpu.sync_copy(ones, ones_v)
        pltpu.sync_copy(src.at[wid], src_v)
        pltpu.sync_copy(dst.at[wid], dst_v)
        plsc.subcore_barrier()

        for k in range(_NIF):
            pltpu.async_copy(table.at[src_v.at[k]], rows[k], gsem[k])

        def step(i, carry):
            base = i * _NBUF
            for k in range(_NBUF):
                j = base + k
                jn = j + _NIF
                bn = (k + _NIF) % _NBUF

                @pl.when(jnp.logical_and(jn < _NCH, j >= _NBUF - _NIF))
                def _():
                    pltpu.make_async_copy(
                        rows[bn], acc.at[dst_v.at[jn - _NBUF]],
                        ssem[bn]).wait()

                @pl.when(jn < _NCH)
                def _():
                    pltpu.async_copy(table.at[src_v.at[jn]], rows[bn],
                                     gsem[bn])

                pltpu.make_async_copy(table.at[src_v.at[j]], rows[k],
                                      gsem[k]).wait()
                pltpu.async_copy(rows[k], acc.at[dst_v.at[j]], ssem[k],
                                 add=True)
                if with_counts:
                    @pl.when(j >= _NBUF)
                    def _():
                        pltpu.make_async_copy(
                            ones_v, acc16.at[dst_v.at[j - _NBUF]],
                            csem).wait()

                    pltpu.async_copy(ones_v, acc16.at[dst_v.at[j]], csem,
                                     add=True)
            return carry

        lax.fori_loop(0, _NCH // _NBUF, step, 0)
        for k in range(_NBUF):
            jt = _NCH - _NBUF + k
            pltpu.make_async_copy(rows[k], acc.at[dst_v.at[jt]],
                                  ssem[k]).wait()
            if with_counts:
                pltpu.make_async_copy(ones_v, acc16.at[dst_v.at[jt]],
                                      csem).wait()
        plsc.subcore_barrier()
        pltpu.sync_copy(acc.at[pl.ds(s * _RPT, _RPT)],
                        out.at[c, pl.ds(s * _RPT, _RPT), pl.ds(0, _DH)])
        if with_counts:
            pltpu.sync_copy(acc16.at[pl.ds(s * _RPT, _RPT)],
                            cout.at[c, pl.ds(s * _RPT, _RPT), pl.ds(0, 16)])

    out_type = jax.ShapeDtypeStruct((_NC, _N_PAD, 2 * _DH), jnp.float32)
    scratch = [pltpu.VMEM_SHARED((_N_PAD, _DH), jnp.float32)]
    if with_counts:
        out_type = (out_type,
                    jax.ShapeDtypeStruct((_NC, _N_PAD, 2 * _DH), jnp.float32))
    scratch += [
        pltpu.VMEM((_NCH, _CH), jnp.int32),
        pltpu.VMEM((_NCH, _CH), jnp.int32),
    ]
    if with_counts:
        scratch += [
            pltpu.VMEM_SHARED((_N_PAD, 16), jnp.float32),
            pltpu.VMEM((_CH, 16), jnp.float32),
        ]
    scratch += [pltpu.VMEM((_CH, _DH), jnp.float32)] * _NBUF
    scratch += [pltpu.SemaphoreType.DMA] * (2 * _NBUF + with_counts)
    return pl.kernel(
        body,
        out_type=out_type,
        mesh=_mesh,
        compiler_params=pltpu.CompilerParams(use_tc_tiling_on_sc=False),
        scratch_types=scratch,
    )


_agg = _make_agg(False)


def _cnt_body(dst, ones, zeros, out, acc, dst_v, ones_v, csem):
    c = lax.axis_index("c")
    s = lax.axis_index("s")
    wid = s * _NC + c
    pltpu.sync_copy(zeros, acc.at[pl.ds(s * _RPT, _RPT)])
    pltpu.sync_copy(dst.at[wid], dst_v)
    pltpu.sync_copy(ones, ones_v)
    plsc.subcore_barrier()

    def step(i, carry):
        base = i * _NBUF
        for k in range(_NBUF):
            j = base + k

            @pl.when(j >= _NBUF)
            def _():
                pltpu.make_async_copy(ones_v, acc.at[dst_v.at[j - _NBUF]],
                                      csem).wait()

            pltpu.async_copy(ones_v, acc.at[dst_v.at[j]], csem, add=True)
        return carry

    lax.fori_loop(0, _NCH // _NBUF, step, 0)
    for k in range(_NBUF):
        pltpu.make_async_copy(ones_v, acc.at[dst_v.at[_NCH - _NBUF + k]],
                              csem).wait()
    plsc.subcore_barrier()
    pltpu.sync_copy(acc.at[pl.ds(s * _RPT, _RPT)],
                    out.at[c, pl.ds(s * _RPT, _RPT), pl.ds(0, 16)])


_cnt = pl.kernel(
    _cnt_body,
    out_type=jax.ShapeDtypeStruct((_NC, _N_PAD, 2 * _DH), jnp.float32),
    mesh=_mesh,
    compiler_params=pltpu.CompilerParams(use_tc_tiling_on_sc=False),
    scratch_types=[
        pltpu.VMEM_SHARED((_N_PAD, 16), jnp.float32),
        pltpu.VMEM((_NCH, _CH), jnp.int32),
        pltpu.VMEM((_CH, 16), jnp.float32),
        pltpu.SemaphoreType.DMA,
    ],
)


def _wfold_kernel(wl0_ref, wr0_ref, wl1_ref, wr1_ref, wl2_ref, wr2_ref,
                  wm_ref, bl0_ref, bl1_ref, bl2_ref, bm_ref, *out_refs):
    wm = wm_ref[...]
    for k, (wl_ref, wr_ref) in enumerate(
            ((wl0_ref, wr0_ref), (wl1_ref, wr1_ref), (wl2_ref, wr2_ref))):
        out_refs[k][:, :_DH] = jnp.dot(wl_ref[...], wm,
                                       preferred_element_type=jnp.float32)
        out_refs[k][:, _DH:] = jnp.dot(wr_ref[...], wm,
                                       preferred_element_type=jnp.float32)
    bm = bm_ref[...]
    for k, b_ref in enumerate((bl0_ref, bl1_ref, bl2_ref)):
        out_refs[3 + k][...] = jnp.dot(b_ref[...], wm,
                                       preferred_element_type=jnp.float32) + bm


def _wfold(wl0, wr0, wl1, wr1, wl2, wr2, wm, bl0, bl1, bl2, bm):
    full = lambda shape: pl.BlockSpec(shape, lambda: (0, 0))
    return pl.pallas_call(
        _wfold_kernel,
        in_specs=[full((_DF, _DH)), full((_DF, _DH))] +
                 [full((_DH, _DH))] * 5 + [full((1, _DH))] * 4,
        out_specs=[full((_DF, 2 * _DH))] + [full((_DH, 2 * _DH))] * 2 +
                  [full((1, _DH))] * 3,
        out_shape=[jax.ShapeDtypeStruct((_DF, 2 * _DH), jnp.float32)] +
                  [jax.ShapeDtypeStruct((_DH, 2 * _DH), jnp.float32)] * 2 +
                  [jax.ShapeDtypeStruct((1, _DH), jnp.float32)] * 3,
    )(wl0, wr0, wl1, wr1, wl2, wr2, wm, bl0, bl1, bl2, bm)


def _mm2_kernel(h_ref, wab_ref, a_ref, r_ref):
    z = jnp.dot(h_ref[...], wab_ref[...], preferred_element_type=jnp.float32)
    a_ref[...] = z[:, :_DH]
    r_ref[...] = z[:, _DH:]


def _mm2(h, wab):
    d = h.shape[1]
    return pl.pallas_call(
        _mm2_kernel,
        grid=(_NBLK,),
        in_specs=[
            pl.BlockSpec((_MB, d), lambda i: (i, 0)),
            pl.BlockSpec((d, 2 * _DH), lambda i: (0, 0)),
        ],
        out_specs=[
            pl.BlockSpec((_MB, _DH), lambda i: (i, 0)),
            pl.BlockSpec((_MB, _DH), lambda i: (i, 0)),
        ],
        out_shape=[
            jax.ShapeDtypeStruct((_N_PAD, _DH), jnp.float32),
            jax.ShapeDtypeStruct((_N_PAD, _DH), jnp.float32),
        ],
    )(h, wab)


def _hcomb(pv, cpv, r, bb):
    psum = (pv[0] + pv[1])[:, :_DH]
    cnt = (cpv[0] + cpv[1])[:, 0:1]
    rcp = 1.0 / jnp.clip(cnt, 1.0, None)
    return jnp.maximum(psum * rcp + bb + r, 0.0)


def _cmb2_kernel(p_ref, cp_ref, r_ref, bb_ref, wab_ref, h_ref, a_ref,
                 r2_ref):
    h = _hcomb(p_ref[...], cp_ref[...], r_ref[...], bb_ref[...])
    h_ref[...] = h
    z = jnp.dot(h, wab_ref[...], preferred_element_type=jnp.float32)
    a_ref[...] = z[:, :_DH]
    r2_ref[...] = z[:, _DH:]


def _cmb2(p, cp, r, bb, wab):
    return pl.pallas_call(
        _cmb2_kernel,
        grid=(_NBLK,),
        in_specs=[
            pl.BlockSpec((_NC, _MB, 2 * _DH), lambda i: (0, i, 0)),
            pl.BlockSpec((_NC, _MB, 2 * _DH), lambda i: (0, i, 0)),
            pl.BlockSpec((_MB, _DH), lambda i: (i, 0)),
            pl.BlockSpec((1, _DH), lambda i: (0, 0)),
            pl.BlockSpec((_DH, 2 * _DH), lambda i: (0, 0)),
        ],
        out_specs=[
            pl.BlockSpec((_MB, _DH), lambda i: (i, 0)),
            pl.BlockSpec((_MB, _DH), lambda i: (i, 0)),
            pl.BlockSpec((_MB, _DH), lambda i: (i, 0)),
        ],
        out_shape=[jax.ShapeDtypeStruct((_N_PAD, _DH), jnp.float32)] * 3,
    )(p, cp, r, bb, wab)


def _pool_kernel(b_ref, h1_ref, h2_ref, p_ref, cp_ref, r_ref, bb_ref,
                 w1_ref, b1_ref, w2_ref, b2_ref, o_ref, acc_ref):
    i = pl.program_id(0)

    @pl.when(i == 0)
    def _():
        acc_ref[...] = jnp.zeros_like(acc_ref)

    h3 = _hcomb(p_ref[...], cp_ref[...], r_ref[...], bb_ref[...])
    bb = b_ref[0, 0, :]
    onehot = (bb[:, None] ==
              lax.broadcasted_iota(jnp.int32, (1, _G), 1)).astype(jnp.float32)
    dn = (((0,), (0,)), ((), ()))
    for k, h in enumerate((h1_ref[...], h2_ref[...], h3)):
        acc_ref[:, k * _DH:(k + 1) * _DH] += lax.dot_general(
            onehot, h, dn, preferred_element_type=jnp.float32)
    acc_ref[:, 3 * _DH:3 * _DH + 1] += jnp.sum(onehot, axis=0)[:, None]

    @pl.when(i == _NBLK - 1)
    def _():
        accv = acc_ref[...]
        rcp = 1.0 / jnp.clip(accv[:, 3 * _DH:3 * _DH + 1], 1.0, None)
        pooled = accv[:, :3 * _DH] * rcp
        z = jnp.dot(pooled, w1_ref[...], preferred_element_type=jnp.float32)
        z = jnp.maximum(z + b1_ref[...], 0.0)
        o_ref[...] = jnp.dot(
            z, w2_ref[...], preferred_element_type=jnp.float32) + b2_ref[...]


def _pool(batch_r, h1, h2, p, cp, r, bb, w1, b1, w2, b2):
    return pl.pallas_call(
        _pool_kernel,
        grid=(_NBLK,),
        in_specs=[
            pl.BlockSpec((1, 1, _MB), lambda i: (i, 0, 0)),
            pl.BlockSpec((_MB, _DH), lambda i: (i, 0)),
            pl.BlockSpec((_MB, _DH), lambda i: (i, 0)),
            pl.BlockSpec((_NC, _MB, 2 * _DH), lambda i: (0, i, 0)),
            pl.BlockSpec((_NC, _MB, 2 * _DH), lambda i: (0, i, 0)),
            pl.BlockSpec((_MB, _DH), lambda i: (i, 0)),
            pl.BlockSpec((1, _DH), lambda i: (0, 0)),
            pl.BlockSpec((3 * _DH, _DH), lambda i: (0, 0)),
            pl.BlockSpec((1, _DH), lambda i: (0, 0)),
            pl.BlockSpec((_DH, _DT), lambda i: (0, 0)),
            pl.BlockSpec((1, _DT), lambda i: (0, 0)),
        ],
        out_specs=pl.BlockSpec((_G, _DT), lambda i: (0, 0)),
        out_shape=jax.ShapeDtypeStruct((_G, _DT), jnp.float32),
        scratch_shapes=[pltpu.VMEM((_G, 3 * _DH + 128), jnp.float32)],
    )(batch_r, h1, h2, p, cp, r, bb, w1, b1, w2, b2)


def kernel(x, edge_index, batch, Wl0, bl0, Wr0, Wl1, bl1, Wr1, Wl2, bl2,
           Wr2, Wm, bm, W1, b1, W2, b2):
    x_p = jnp.pad(x, ((0, _N_PAD - _N), (0, 0)))
    src = edge_index[0].astype(jnp.int32)
    dst = edge_index[1].astype(jnp.int32)
    npad = _E_PAD - _E
    ar = jnp.arange(npad, dtype=jnp.int32)
    src_p = jnp.concatenate([src, (ar * 97) % _N]).reshape(_NW, _NCH, _CH)
    dst_p = jnp.concatenate([dst, _N + ar % (_N_PAD - _N)]).reshape(_NW, _NCH, _CH)
    zeros64 = jnp.zeros((_RPT, _DH), jnp.float32)
    zeros16 = jnp.zeros((_RPT, 16), jnp.float32)
    ones16 = jnp.ones((_CH, 16), jnp.float32)
    batch_r = jnp.pad(batch.astype(jnp.int32), (0, _N_PAD - _N),
                      constant_values=_G).reshape(_NBLK, 1, _MB)
    blr0, blr1, blr2 = (b.reshape(1, _DH) for b in (bl0, bl1, bl2))
    bmr = bm.reshape(1, _DH)
    b1r = b1.reshape(1, _DH)
    b2r = b2.reshape(1, _DT)

    (wab0, wab1, wab2, bb0, bb1, bb2) = _wfold(
        Wl0, Wr0, Wl1, Wr1, Wl2, Wr2, Wm, blr0, blr1, blr2, bmr)

    cp = _cnt(dst_p, ones16, zeros16)
    a0, r0 = _mm2(x_p, wab0)
    p0 = _agg(a0, src_p, dst_p, zeros64)
    h1, a1, r1 = _cmb2(p0, cp, r0, bb0, wab1)

    p1 = _agg(a1, src_p, dst_p, zeros64)
    h2, a2, r2 = _cmb2(p1, cp, r1, bb1, wab2)

    p2 = _agg(a2, src_p, dst_p, zeros64)

    return _pool(batch_r, h1, h2, p2, cp, r2, bb2, W1, b1r, W2, b2r)
